# baseline (speedup 1.0000x reference)
import functools
import jax
import jax.numpy as jnp
import numpy as np
from jax import lax
from jax.experimental import pallas as pl
from jax.experimental.pallas import tpu as pltpu
from jax.experimental.pallas import tpu_sc as plsc

_N = 20000
_C = 91
_IMG_H = 800.0
_IMG_W = 1066.0
_SCORE_THRESH = 0.05
_NMS_THRESH = 0.5
_DET = 100
_KPRE = 2000
_CLIP = float(np.log(1000.0 / 16.0))

_BR = 400

_TOT = _N * 90
_TILES = 32
_PERT = 56256
_LAST = _TOT - 31 * _PERT
_VREGS = _PERT // 16
_INT_MIN = -2147483648
_BASE13 = 0x3D4CCCCE >> 13
_NB1 = 4608


def _dense_body(lg_ref, dx_ref, dy_ref, dw_ref, dh_ref, pr_ref,
                key_ref, bx1_ref, by1_ref, bx2_ref, by2_ref):
    logits = lg_ref[...]
    m = jnp.max(logits, axis=-1, keepdims=True)
    e = jnp.exp(logits - m)
    scores = (e / jnp.sum(e, axis=-1, keepdims=True))[:, 1:]

    px1 = pr_ref[:, 0]
    py1 = pr_ref[:, 1]
    px2 = pr_ref[:, 2]
    py2 = pr_ref[:, 3]
    widths = px2 - px1
    heights = py2 - py1
    ctr_x = px1 + 0.5 * widths
    ctr_y = py1 + 0.5 * heights

    dx = dx_ref[...] * (1.0 / 10.0)
    dy = dy_ref[...] * (1.0 / 10.0)
    dw = jnp.minimum(dw_ref[...] * (1.0 / 5.0), _CLIP)
    dh = jnp.minimum(dh_ref[...] * (1.0 / 5.0), _CLIP)

    pcx = dx * widths[:, None] + ctr_x[:, None]
    pcy = dy * heights[:, None] + ctr_y[:, None]
    pw = jnp.exp(dw) * widths[:, None]
    ph = jnp.exp(dh) * heights[:, None]

    bx1 = jnp.clip(pcx - 0.5 * pw, 0.0, _IMG_W)
    by1 = jnp.clip(pcy - 0.5 * ph, 0.0, _IMG_H)
    bx2 = jnp.clip(pcx + 0.5 * pw, 0.0, _IMG_W)
    by2 = jnp.clip(pcy + 0.5 * ph, 0.0, _IMG_H)

    ws = bx2 - bx1
    hs = by2 - by1
    valid = (scores > _SCORE_THRESH) & (ws >= 1e-2) & (hs >= 1e-2)
    masked = jnp.where(valid, scores, -1.0)
    v = lax.bitcast_convert_type(masked, jnp.int32)
    key_ref[...] = jnp.where(v >= 0, v, v ^ 0x7FFFFFFF)
    bx1_ref[...] = bx1
    by1_ref[...] = by1
    bx2_ref[...] = bx2
    by2_ref[...] = by2


def _dense_stage(class_logits, box_regression, proposals):
    reg = box_regression
    dx = reg[:, 4::4]
    dy = reg[:, 5::4]
    dw = reg[:, 6::4]
    dh = reg[:, 7::4]

    grid = (_N // _BR,)
    row_spec = pl.BlockSpec((_BR, 90), lambda i: (i, 0))
    fshape = jax.ShapeDtypeStruct((_N, 90), jnp.float32)
    ishape = jax.ShapeDtypeStruct((_N, 90), jnp.int32)
    return pl.pallas_call(
        _dense_body,
        grid=grid,
        in_specs=[
            pl.BlockSpec((_BR, _C), lambda i: (i, 0)),
            row_spec, row_spec, row_spec, row_spec,
            pl.BlockSpec((_BR, 4), lambda i: (i, 0)),
        ],
        out_specs=[row_spec] * 5,
        out_shape=[ishape, fshape, fshape, fshape, fshape],
    )(class_logits, dx, dy, dw, dh, proposals)


def _sc_mesh():
    return plsc.VectorSubcoreMesh(core_axis_name="c", subcore_axis_name="s")


def _wid():
    return lax.axis_index("s") * 2 + lax.axis_index("c")


def _load_chunk(keys_hbm, buf, wid):
    base = wid * _PERT

    @pl.when(wid == _TILES - 1)
    def _():
        mn = jnp.full((16,), _INT_MIN, jnp.int32)

        def zt(i, _):
            buf[pl.ds(_LAST + i * 16, 16)] = mn
            return 0

        lax.fori_loop(0, (_PERT - _LAST) // 16, zt, 0)
        pltpu.sync_copy(keys_hbm.at[pl.ds(base, _LAST)],
                        buf.at[pl.ds(0, _LAST)])

    @pl.when(wid < _TILES - 1)
    def _():
        pltpu.sync_copy(keys_hbm.at[pl.ds(base, _PERT)], buf)


def _bin1(v):
    return jnp.clip((v >> 13) - (_BASE13 - 1), 0, _NB1 - 1)


def _sc_hist1_body(keys_hbm, hist_hbm, buf, lh):
    wid = _wid()
    zeros16 = jnp.zeros((16,), jnp.int32)

    def z(i, _):
        lh[pl.ds(i * 16, 16)] = zeros16
        return 0

    lax.fori_loop(0, _NB1, z, 0)
    _load_chunk(keys_hbm, buf, wid)
    iota = lax.iota(jnp.int32, 16)
    ones = jnp.ones((16,), jnp.int32)

    def h(i, _):
        v = buf[pl.ds(i * 16, 16)]
        idx = iota * _NB1 + _bin1(v)
        plsc.addupdate_scatter(lh, [idx], ones)
        return 0

    lax.fori_loop(0, _VREGS, h, 0)

    def r(g, _):
        acc = lh[pl.ds(g * 16, 16)]
        for l in range(1, 16):
            acc = acc + lh[pl.ds(l * _NB1 + g * 16, 16)]
        buf[pl.ds(g * 16, 16)] = acc
        return 0

    lax.fori_loop(0, _NB1 // 16, r, 0)
    pltpu.sync_copy(buf.at[pl.ds(0, _NB1)], hist_hbm.at[wid])


def _sc_hist1(keys_flat):
    return pl.kernel(
        _sc_hist1_body,
        out_type=jax.ShapeDtypeStruct((_TILES, _NB1), jnp.int32),
        mesh=_sc_mesh(),
        compiler_params=pltpu.CompilerParams(needs_layout_passes=False),
        scratch_types=[
            pltpu.VMEM((_PERT,), jnp.int32),
            pltpu.VMEM((16 * _NB1,), jnp.int32),
        ],
    )(keys_flat)


def _sc_hist2_body(keys_hbm, b1_hbm, hist_hbm, buf, lh, bv):
    wid = _wid()
    zeros16 = jnp.zeros((16,), jnp.int32)

    def z(i, _):
        lh[pl.ds(i * 16, 16)] = zeros16
        return 0

    lax.fori_loop(0, 256, z, 0)
    pltpu.sync_copy(b1_hbm.at[0], bv)
    _load_chunk(keys_hbm, buf, wid)
    iota = lax.iota(jnp.int32, 16)
    ones = jnp.ones((16,), jnp.int32)
    vb1 = bv[pl.ds(0, 16)]

    def h(i, _):
        v = buf[pl.ds(i * 16, 16)]
        m = _bin1(v) == vb1
        idx = iota * 256 + ((v >> 5) & 0xFF)
        plsc.addupdate_scatter(lh, [idx], ones, mask=m)
        return 0

    lax.fori_loop(0, _VREGS, h, 0)

    def r(g, _):
        acc = lh[pl.ds(g * 16, 16)]
        for l in range(1, 16):
            acc = acc + lh[pl.ds(l * 256 + g * 16, 16)]
        buf[pl.ds(g * 16, 16)] = acc
        return 0

    lax.fori_loop(0, 16, r, 0)
    pltpu.sync_copy(buf.at[pl.ds(0, 256)], hist_hbm.at[wid])


def _sc_hist2(keys_flat, b1_vec):
    return pl.kernel(
        _sc_hist2_body,
        out_type=jax.ShapeDtypeStruct((_TILES, 256), jnp.int32),
        mesh=_sc_mesh(),
        compiler_params=pltpu.CompilerParams(needs_layout_passes=False),
        scratch_types=[
            pltpu.VMEM((_PERT,), jnp.int32),
            pltpu.VMEM((16 * 256,), jnp.int32),
            pltpu.VMEM((128,), jnp.int32),
        ],
    )(keys_flat, b1_vec)


def _sc_hist3_body(keys_hbm, b1_hbm, b2_hbm, hist_hbm, buf, lh, bv1, bv2):
    wid = _wid()
    zeros16 = jnp.zeros((16,), jnp.int32)

    def z(i, _):
        lh[pl.ds(i * 16, 16)] = zeros16
        return 0

    lax.fori_loop(0, 32, z, 0)
    pltpu.sync_copy(b1_hbm.at[0], bv1)
    pltpu.sync_copy(b2_hbm.at[0], bv2)
    _load_chunk(keys_hbm, buf, wid)
    iota = lax.iota(jnp.int32, 16)
    ones = jnp.ones((16,), jnp.int32)
    vb1 = bv1[pl.ds(0, 16)]
    vb2 = bv2[pl.ds(0, 16)]

    def h(i, _):
        v = buf[pl.ds(i * 16, 16)]
        m = (_bin1(v) == vb1) & (((v >> 5) & 0xFF) == vb2)
        idx = iota * 32 + (v & 0x1F)
        plsc.addupdate_scatter(lh, [idx], ones, mask=m)
        return 0

    lax.fori_loop(0, _VREGS, h, 0)

    def r(g, _):
        acc = lh[pl.ds(g * 16, 16)]
        for l in range(1, 16):
            acc = acc + lh[pl.ds(l * 32 + g * 16, 16)]
        buf[pl.ds(g * 16, 16)] = acc
        return 0

    lax.fori_loop(0, 2, r, 0)

    def zr(g, _):
        buf[pl.ds(32 + g * 16, 16)] = zeros16
        return 0

    lax.fori_loop(0, 6, zr, 0)
    pltpu.sync_copy(buf.at[pl.ds(0, 128)], hist_hbm.at[wid])


def _sc_hist3(keys_flat, b1_vec, b2_vec):
    return pl.kernel(
        _sc_hist3_body,
        out_type=jax.ShapeDtypeStruct((_TILES, 128), jnp.int32),
        mesh=_sc_mesh(),
        compiler_params=pltpu.CompilerParams(needs_layout_passes=False),
        scratch_types=[
            pltpu.VMEM((_PERT,), jnp.int32),
            pltpu.VMEM((16 * 32,), jnp.int32),
            pltpu.VMEM((128,), jnp.int32),
            pltpu.VMEM((128,), jnp.int32),
        ],
    )(keys_flat, b1_vec, b2_vec)


def _suffix_excl(g):
    R, L = g.shape
    src = lax.broadcasted_iota(jnp.int32, (L, L), 0)
    dst = lax.broadcasted_iota(jnp.int32, (L, L), 1)
    upper = (src > dst).astype(jnp.float32)
    w = lax.dot_general(g, upper, (((1,), (0,)), ((), ())),
                        preferred_element_type=jnp.float32)
    t = jnp.sum(g, axis=1, keepdims=True)
    rs = lax.broadcasted_iota(jnp.int32, (R, R), 0)
    rd = lax.broadcasted_iota(jnp.int32, (R, R), 1)
    later_rows = (rd > rs).astype(jnp.float32)
    carry = lax.dot_general(later_rows, t, (((1,), (0,)), ((), ())),
                            preferred_element_type=jnp.float32)
    return w + carry


def _crossing(g, suf, target):
    R, L = g.shape
    mask = (suf < target) & (suf + g >= target)
    bi = (lax.broadcasted_iota(jnp.int32, (R, L), 0) * L +
          lax.broadcasted_iota(jnp.int32, (R, L), 1))
    bstar = jnp.sum(jnp.where(mask, bi, 0))
    above = jnp.sum(jnp.where(mask, suf, 0.0)).astype(jnp.int32)
    return bstar, above


def _r1_body(h_ref, out_ref):
    h = h_ref[...].astype(jnp.float32)
    g = jnp.sum(h, axis=0)
    suf = _suffix_excl(g)
    bstar, a1 = _crossing(g, suf, float(_KPRE))
    lane = lax.broadcasted_iota(jnp.int32, (1, 128), 1)
    out_ref[...] = jnp.where(lane == 0, bstar, jnp.where(lane == 1, a1, 0))


def _r2_body(h_ref, r1_ref, out_ref):
    h = h_ref[...].astype(jnp.float32)
    a1 = r1_ref[0, 1]
    g = jnp.sum(h, axis=0)
    suf = _suffix_excl(g)
    b2, a2 = _crossing(g, suf, (_KPRE - a1).astype(jnp.float32))
    lane = lax.broadcasted_iota(jnp.int32, (1, 128), 1)
    out_ref[...] = jnp.where(lane == 0, b2, jnp.where(lane == 1, a2, 0))


def _r3_body(h1_ref, h2_ref, h3_ref, r1_ref, r2_ref,
             kstar_ref, o_ref, quota_ref, ts_ref):
    b1s = r1_ref[0, 0]
    a1 = r1_ref[0, 1]
    b2s = r2_ref[0, 0]
    a2 = r2_ref[0, 1]
    h3 = h3_ref[...].astype(jnp.float32)
    g3 = jnp.sum(h3, axis=0, keepdims=True)
    suf3 = _suffix_excl(g3)
    need2 = (_KPRE - a1 - a2).astype(jnp.float32)
    b3s, a3 = _crossing(g3, suf3, need2)
    a_total = a1 + a2 + a3
    tcnt = _KPRE - a_total

    key_normal = ((b1s - 1 + _BASE13) << 13) | (b2s << 5) | b3s
    key_bin0 = jnp.where(b2s == 0xFF, jnp.int32(-1065353217),
                         jnp.int32(_INT_MIN))
    kstar = jnp.where(b1s >= 1, key_normal, key_bin0)

    h1 = h1_ref[...].astype(jnp.float32)
    bi1 = (lax.broadcasted_iota(jnp.int32, (36, 128), 0) * 128 +
           lax.broadcasted_iota(jnp.int32, (36, 128), 1))
    g1_t = jnp.sum(h1 * (bi1 > b1s).astype(jnp.float32)[None], axis=(1, 2))
    h2 = h2_ref[...].astype(jnp.float32)
    bi2 = (lax.broadcasted_iota(jnp.int32, (2, 128), 0) * 128 +
           lax.broadcasted_iota(jnp.int32, (2, 128), 1))
    g2_t = jnp.sum(h2 * (bi2 > b2s).astype(jnp.float32)[None], axis=(1, 2))
    bi3 = lax.broadcasted_iota(jnp.int32, (_TILES, 128), 1)
    g3_t = jnp.sum(h3 * (bi3 > b3s).astype(jnp.float32), axis=1)
    c_t = jnp.sum(h3 * (bi3 == b3s).astype(jnp.float32), axis=1)
    g_t = (g1_t + g2_t + g3_t).reshape(1, _TILES)
    c_t = c_t.reshape(1, _TILES)

    ts_i = lax.broadcasted_iota(jnp.int32, (_TILES, _TILES), 0)
    td_i = lax.broadcasted_iota(jnp.int32, (_TILES, _TILES), 1)
    lower = (ts_i < td_i).astype(jnp.float32)
    o_t = lax.dot_general(g_t, lower, (((1,), (0,)), ((), ())),
                          preferred_element_type=jnp.float32)
    p_t = lax.dot_general(c_t, lower, (((1,), (0,)), ((), ())),
                          preferred_element_type=jnp.float32)
    tf = tcnt.astype(jnp.float32)
    quota_t = jnp.clip(tf - p_t, 0.0, c_t)
    tiestart_t = a_total.astype(jnp.float32) + jnp.minimum(p_t, tf)

    kstar_ref[...] = jnp.broadcast_to(kstar, (1, 128))
    zpad = jnp.zeros((1, 128 - _TILES), jnp.float32)
    o_ref[...] = jnp.concatenate([o_t, zpad], axis=1).astype(jnp.int32)
    quota_ref[...] = jnp.concatenate([quota_t, zpad], axis=1).astype(jnp.int32)
    ts_ref[...] = jnp.concatenate([tiestart_t, zpad], axis=1).astype(jnp.int32)


def _reduce_kernels(h1, h2, h3):
    h1_3d = h1.reshape(_TILES, 36, 128)
    h2_3d = h2.reshape(_TILES, 2, 128)
    r1 = pl.pallas_call(
        _r1_body, out_shape=jax.ShapeDtypeStruct((1, 128), jnp.int32),
    )(h1_3d)
    r2 = pl.pallas_call(
        _r2_body, out_shape=jax.ShapeDtypeStruct((1, 128), jnp.int32),
    )(h2_3d, r1)
    kstar, o_t, quota_t, ts_t = pl.pallas_call(
        _r3_body,
        out_shape=[
            jax.ShapeDtypeStruct((1, 16), jnp.int32),
            jax.ShapeDtypeStruct((1, _TILES), jnp.int32),
            jax.ShapeDtypeStruct((1, _TILES), jnp.int32),
            jax.ShapeDtypeStruct((1, _TILES), jnp.int32),
        ],
    )(h1_3d, h2_3d, h3, r1, r2)
    return r1, r2, kstar, o_t, quota_t, ts_t


def _sc_compact_body(keys_hbm, kvec_hbm, o_hbm, q_hbm, ts_hbm,
                     outk_hbm, outi_hbm,
                     buf, gbk, gbi, tbi, pos, kv16, sco, scq, scts, sem):
    wid = _wid()
    base = wid * _PERT
    pltpu.sync_copy(kvec_hbm.at[0], kv16)
    pltpu.sync_copy(o_hbm.at[0], sco)
    pltpu.sync_copy(q_hbm.at[0], scq)
    pltpu.sync_copy(ts_hbm.at[0], scts)
    _load_chunk(keys_hbm, buf, wid)
    iota = lax.iota(jnp.int32, 16)
    kv = kv16[pl.ds(0, 16)]

    def scal(vec_ref):
        half = wid // 16
        lane = wid % 16
        vec = vec_ref[pl.ds(half * 16, 16)]
        return lax.reduce_max(jnp.where(iota == lane, vec, 0), axes=(0,))

    def step(i, carry):
        ng, nt = carry
        v = buf[pl.ds(i * 16, 16)]
        gm = v > kv
        tm = v == kv
        idxv = (base + i * 16) + iota
        plsc.store_compressed(gbk.at[pl.ds(ng, 16)], v, mask=gm)
        plsc.store_compressed(gbi.at[pl.ds(ng, 16)], idxv, mask=gm)
        tm2 = tm & (nt < 2032)
        plsc.store_compressed(tbi.at[pl.ds(nt, 16)], idxv, mask=tm2)
        cg = lax.reduce_max(plsc.all_reduce_population_count(gm), axes=(0,))
        ct = lax.reduce_max(plsc.all_reduce_population_count(tm2), axes=(0,))
        return ng + cg, nt + ct

    ng, nt = lax.fori_loop(0, _VREGS, step, (jnp.int32(0), jnp.int32(0)))

    my_o = scal(sco)
    my_q = scal(scq)
    my_ts = scal(scts)
    dump = 2048 + wid * 2

    def mkpos(j, start, count):
        r = j * 16 + iota
        return jnp.where(r < count, start + r, dump)

    def p1(j, _):
        pos[pl.ds(j * 16, 16)] = mkpos(j, my_o, ng)
        return 0

    lax.fori_loop(0, 129, p1, 0)
    pltpu.async_copy(gbk, outk_hbm.at[pos], sem).wait()
    pltpu.async_copy(gbi, outi_hbm.at[pos], sem).wait()

    def p2(j, _):
        pos[pl.ds(j * 16, 16)] = mkpos(j, my_ts, my_q)
        return 0

    lax.fori_loop(0, 129, p2, 0)

    def fillk(j, _):
        gbk[pl.ds(j * 16, 16)] = kv
        return 0

    lax.fori_loop(0, 129, fillk, 0)
    pltpu.async_copy(tbi, outi_hbm.at[pos], sem).wait()
    pltpu.async_copy(gbk, outk_hbm.at[pos], sem).wait()

    @pl.when(wid == 0)
    def _():
        mn = jnp.full((16,), _INT_MIN, jnp.int32)
        zz = jnp.zeros((16,), jnp.int32)
        for j in range(3):
            gbi[pl.ds(j * 16, 16)] = mn
            tbi[pl.ds(j * 16, 16)] = zz
        pltpu.sync_copy(gbi.at[pl.ds(0, 48)],
                        outk_hbm.at[pl.ds(_KPRE, 48)])
        pltpu.sync_copy(tbi.at[pl.ds(0, 48)],
                        outi_hbm.at[pl.ds(_KPRE, 48)])


def _sc_compact(keys_flat, kstar_vec, o_t, quota_t, ts_t):
    return pl.kernel(
        _sc_compact_body,
        out_type=[
            jax.ShapeDtypeStruct((4096,), jnp.int32),
            jax.ShapeDtypeStruct((4096,), jnp.int32),
        ],
        mesh=_sc_mesh(),
        compiler_params=pltpu.CompilerParams(needs_layout_passes=False),
        scratch_types=[
            pltpu.VMEM((_PERT,), jnp.int32),
            pltpu.VMEM((2064,), jnp.int32),
            pltpu.VMEM((2064,), jnp.int32),
            pltpu.VMEM((2064,), jnp.int32),
            pltpu.VMEM((2064,), jnp.int32),
            pltpu.VMEM((128,), jnp.int32),
            pltpu.VMEM((128,), jnp.int32),
            pltpu.VMEM((128,), jnp.int32),
            pltpu.VMEM((128,), jnp.int32),
            pltpu.SemaphoreType.DMA,
        ],
    )(keys_flat, kstar_vec, o_t, quota_t, ts_t)


def _sort_body(k_ref, v_ref, ok_ref, ov_ref):
    keys = k_ref[...]
    vals = v_ref[...]
    i = lax.broadcasted_iota(jnp.int32, (1, 2048), 1)
    for size_p in range(1, 12):
        size = 1 << size_p
        dirm = (i & size) == 0
        for j_p in range(size_p - 1, -1, -1):
            j = 1 << j_p
            upperm = (i & j) != 0
            pk = jnp.where(upperm, pltpu.roll(keys, j, 1),
                           pltpu.roll(keys, 2048 - j, 1))
            pv = jnp.where(upperm, pltpu.roll(vals, j, 1),
                           pltpu.roll(vals, 2048 - j, 1))
            pf = (pk > keys) | ((pk == keys) & (pv < vals))
            take = pf ^ upperm ^ dirm ^ True
            keys = jnp.where(take, pk, keys)
            vals = jnp.where(take, pv, vals)
    ok_ref[...] = keys
    ov_ref[...] = vals


def _sort2048(keys, vals):
    return pl.pallas_call(
        _sort_body,
        out_shape=[jax.ShapeDtypeStruct((1, 2048), jnp.int32)] * 2,
    )(keys.reshape(1, 2048), vals.reshape(1, 2048))


_KP = 2048
_NB = 128
_NBLK = _KP // _NB


def _nms_body(x1c, y1c, x2c, y2c, x1r, y1r, x2r, y2r, keep_ref,
              iou_s, bb_s, kb_s):
    ax1 = x1r[...]
    ay1 = y1r[...]
    ax2 = x2r[...]
    ay2 = y2r[...]
    area_r = jnp.maximum(ax2 - ax1, 0.0) * jnp.maximum(ay2 - ay1, 0.0)
    keep_ref[...] = jnp.ones((1, _KP), jnp.float32)
    lane = lax.broadcasted_iota(jnp.int32, (1, _NB), 1)
    col = lax.broadcasted_iota(jnp.int32, (1, _KP), 1)

    for bi in range(_NBLK):
        s = bi * _NB
        xb1 = x1c[s:s + _NB, :]
        yb1 = y1c[s:s + _NB, :]
        xb2 = x2c[s:s + _NB, :]
        yb2 = y2c[s:s + _NB, :]
        area_b = jnp.maximum(xb2 - xb1, 0.0) * jnp.maximum(yb2 - yb1, 0.0)
        xx1 = jnp.maximum(xb1, ax1)
        yy1 = jnp.maximum(yb1, ay1)
        xx2 = jnp.minimum(xb2, ax2)
        yy2 = jnp.minimum(yb2, ay2)
        inter = jnp.maximum(xx2 - xx1, 0.0) * jnp.maximum(yy2 - yy1, 0.0)
        iou = inter / (area_b + area_r - inter + 1e-9)
        hit = (iou > _NMS_THRESH).astype(jnp.float32)
        iou_s[...] = hit
        bb_s[...] = hit[:, s:s + _NB]
        kb_s[...] = keep_ref[0:1, s:s + _NB]

        def intra(i, _):
            row = bb_s[pl.ds(i, 1), :]
            kb = kb_s[...]
            ki = jnp.max(jnp.where(lane == i, kb, 0.0))
            sup = (row > 0.0) & (lane > i) & (ki > 0.0)
            kb_s[...] = jnp.where(sup, 0.0, kb)
            return 0

        lax.fori_loop(0, _NB, intra, 0, unroll=False)

        kept = kb_s[...]
        keep_ref[0:1, s:s + _NB] = kept
        sup_all = lax.dot_general(
            kept, iou_s[...],
            dimension_numbers=(((1,), (0,)), ((), ())),
            preferred_element_type=jnp.float32)
        later = col >= (s + _NB)
        keep_ref[...] = jnp.where((sup_all > 0.0) & later, 0.0,
                                  keep_ref[...])


def _nms_keep_pallas(boxes):
    pad = _KP - _KPRE
    b = jnp.pad(boxes, ((0, pad), (0, 0)))
    cols = [b[:, i:i + 1] for i in range(4)]
    rows = [b[:, i].reshape(1, _KP) for i in range(4)]
    keep = pl.pallas_call(
        _nms_body,
        out_shape=jax.ShapeDtypeStruct((1, _KP), jnp.float32),
        scratch_shapes=[
            pltpu.VMEM((_NB, _KP), jnp.float32),
            pltpu.VMEM((_NB, _NB), jnp.float32),
            pltpu.VMEM((1, _NB), jnp.float32),
        ],
    )(*cols, *rows)
    return keep[0, :_KPRE] > 0.0


@jax.jit
def kernel(class_logits, box_regression, proposals):
    keys, bx1, by1, bx2, by2 = _dense_stage(
        class_logits, box_regression, proposals)
    keys_flat = keys.reshape(-1)

    h1 = _sc_hist1(keys_flat)
    r1 = pl.pallas_call(
        _r1_body, out_shape=jax.ShapeDtypeStruct((1, 128), jnp.int32),
    )(h1.reshape(_TILES, 36, 128))
    b1_vec = jnp.broadcast_to(r1[0:1, 0:1], (1, 128))
    h2 = _sc_hist2(keys_flat, b1_vec)
    r2 = pl.pallas_call(
        _r2_body, out_shape=jax.ShapeDtypeStruct((1, 128), jnp.int32),
    )(h2.reshape(_TILES, 2, 128), r1)
    b2_vec = jnp.broadcast_to(r2[0:1, 0:1], (1, 128))
    h3 = _sc_hist3(keys_flat, b1_vec, b2_vec)
    kstar, o_t, quota_t, ts_t = pl.pallas_call(
        _r3_body,
        out_shape=[jax.ShapeDtypeStruct((1, 128), jnp.int32)] * 4,
    )(h1.reshape(_TILES, 36, 128), h2.reshape(_TILES, 2, 128), h3, r1, r2)

    outk, outi = _sc_compact(keys_flat, kstar, o_t, quota_t, ts_t)
    sk, sv = _sort2048(outk[:2048], outi[:2048])

    top_idx = sv[0, :_KPRE]
    skk = sk[0, :_KPRE]
    vbits = jnp.where(skk >= 0, skk, skk ^ 0x7FFFFFFF)
    top_vals = lax.bitcast_convert_type(vbits, jnp.float32)

    sel_x1 = bx1.reshape(-1)[top_idx]
    sel_y1 = by1.reshape(-1)[top_idx]
    sel_x2 = bx2.reshape(-1)[top_idx]
    sel_y2 = by2.reshape(-1)[top_idx]
    sel_boxes = jnp.stack([sel_x1, sel_y1, sel_x2, sel_y2], axis=1)
    sel_labels = (top_idx % 90) + 1
    max_coord = jnp.max(sel_boxes)
    offsets = sel_labels.astype(jnp.float32) * (max_coord + 1.0)
    keep = _nms_keep_pallas(sel_boxes + offsets[:, None])
    final_scores = jnp.where(keep & (top_vals > _SCORE_THRESH), top_vals, -1.0)
    fvals, fidx = jax.lax.top_k(final_scores, _DET)
    out = jnp.concatenate([sel_boxes[fidx], fvals[:, None]], axis=1)
    return jnp.where((fvals > _SCORE_THRESH)[:, None], out, 0.0)


# speedup vs baseline: 37.4592x; 37.4592x over previous
import functools
import jax
import jax.numpy as jnp
import numpy as np
from jax import lax
from jax.experimental import pallas as pl
from jax.experimental.pallas import tpu as pltpu
from jax.experimental.pallas import tpu_sc as plsc

_N = 20000
_C = 91
_IMG_H = 800.0
_IMG_W = 1066.0
_SCORE_THRESH = 0.05
_NMS_THRESH = 0.5
_DET = 100
_KPRE = 2000
_CLIP = float(np.log(1000.0 / 16.0))

_BR = 400

_TOT = _N * 90
_TILES = 32
_PERT = 56256
_LAST = _TOT - 31 * _PERT
_VREGS = _PERT // 16
_INT_MIN = -2147483648
_BASE13 = 0x3D4CCCCE >> 13
_NB1 = 4608


def _dense_body(lg_ref, dx_ref, dy_ref, dw_ref, dh_ref, pr_ref,
                key_ref, bx1_ref, by1_ref, bx2_ref, by2_ref):
    logits = lg_ref[...]
    m = jnp.max(logits, axis=-1, keepdims=True)
    e = jnp.exp(logits - m)
    scores = (e / jnp.sum(e, axis=-1, keepdims=True))[:, 1:]

    px1 = pr_ref[:, 0]
    py1 = pr_ref[:, 1]
    px2 = pr_ref[:, 2]
    py2 = pr_ref[:, 3]
    widths = px2 - px1
    heights = py2 - py1
    ctr_x = px1 + 0.5 * widths
    ctr_y = py1 + 0.5 * heights

    dx = dx_ref[...] * (1.0 / 10.0)
    dy = dy_ref[...] * (1.0 / 10.0)
    dw = jnp.minimum(dw_ref[...] * (1.0 / 5.0), _CLIP)
    dh = jnp.minimum(dh_ref[...] * (1.0 / 5.0), _CLIP)

    pcx = dx * widths[:, None] + ctr_x[:, None]
    pcy = dy * heights[:, None] + ctr_y[:, None]
    pw = jnp.exp(dw) * widths[:, None]
    ph = jnp.exp(dh) * heights[:, None]

    bx1 = jnp.clip(pcx - 0.5 * pw, 0.0, _IMG_W)
    by1 = jnp.clip(pcy - 0.5 * ph, 0.0, _IMG_H)
    bx2 = jnp.clip(pcx + 0.5 * pw, 0.0, _IMG_W)
    by2 = jnp.clip(pcy + 0.5 * ph, 0.0, _IMG_H)

    ws = bx2 - bx1
    hs = by2 - by1
    valid = (scores > _SCORE_THRESH) & (ws >= 1e-2) & (hs >= 1e-2)
    masked = jnp.where(valid, scores, -1.0)
    v = lax.bitcast_convert_type(masked, jnp.int32)
    key_ref[...] = jnp.where(v >= 0, v, v ^ 0x7FFFFFFF)
    bx1_ref[...] = bx1
    by1_ref[...] = by1
    bx2_ref[...] = bx2
    by2_ref[...] = by2


def _dense_stage(class_logits, box_regression, proposals):
    reg = box_regression
    dx = reg[:, 4::4]
    dy = reg[:, 5::4]
    dw = reg[:, 6::4]
    dh = reg[:, 7::4]

    grid = (_N // _BR,)
    row_spec = pl.BlockSpec((_BR, 90), lambda i: (i, 0))
    fshape = jax.ShapeDtypeStruct((_N, 90), jnp.float32)
    ishape = jax.ShapeDtypeStruct((_N, 90), jnp.int32)
    return pl.pallas_call(
        _dense_body,
        grid=grid,
        in_specs=[
            pl.BlockSpec((_BR, _C), lambda i: (i, 0)),
            row_spec, row_spec, row_spec, row_spec,
            pl.BlockSpec((_BR, 4), lambda i: (i, 0)),
        ],
        out_specs=[row_spec] * 5,
        out_shape=[ishape, fshape, fshape, fshape, fshape],
    )(class_logits, dx, dy, dw, dh, proposals)


def _sc_mesh():
    return plsc.VectorSubcoreMesh(core_axis_name="c", subcore_axis_name="s")


def _wid():
    return lax.axis_index("s") * 2 + lax.axis_index("c")


def _load_chunk(keys_hbm, buf, wid):
    base = wid * _PERT

    @pl.when(wid == _TILES - 1)
    def _():
        mn = jnp.full((16,), _INT_MIN, jnp.int32)

        def zt(i, _):
            buf[pl.ds(_LAST + i * 16, 16)] = mn
            return 0

        lax.fori_loop(0, (_PERT - _LAST) // 16, zt, 0)
        pltpu.sync_copy(keys_hbm.at[pl.ds(base, _LAST)],
                        buf.at[pl.ds(0, _LAST)])

    @pl.when(wid < _TILES - 1)
    def _():
        pltpu.sync_copy(keys_hbm.at[pl.ds(base, _PERT)], buf)


def _bin1(v):
    return jnp.clip((v >> 13) - (_BASE13 - 1), 0, _NB1 - 1)


def _sc_hist1_body(keys_hbm, hist_hbm, buf, lh):
    wid = _wid()
    zeros16 = jnp.zeros((16,), jnp.int32)

    def z(i, _):
        lh[pl.ds(i * 16, 16)] = zeros16
        return 0

    lax.fori_loop(0, _NB1, z, 0)
    _load_chunk(keys_hbm, buf, wid)
    iota = lax.iota(jnp.int32, 16)
    ones = jnp.ones((16,), jnp.int32)

    def h(i, _):
        v = buf[pl.ds(i * 16, 16)]
        idx = iota * _NB1 + _bin1(v)
        plsc.addupdate_scatter(lh, [idx], ones)
        return 0

    lax.fori_loop(0, _VREGS, h, 0)

    def r(g, _):
        acc = lh[pl.ds(g * 16, 16)]
        for l in range(1, 16):
            acc = acc + lh[pl.ds(l * _NB1 + g * 16, 16)]
        buf[pl.ds(g * 16, 16)] = acc
        return 0

    lax.fori_loop(0, _NB1 // 16, r, 0)
    pltpu.sync_copy(buf.at[pl.ds(0, _NB1)], hist_hbm.at[wid])


def _sc_hist1(keys_flat):
    return pl.kernel(
        _sc_hist1_body,
        out_type=jax.ShapeDtypeStruct((_TILES, _NB1), jnp.int32),
        mesh=_sc_mesh(),
        compiler_params=pltpu.CompilerParams(needs_layout_passes=False),
        scratch_types=[
            pltpu.VMEM((_PERT,), jnp.int32),
            pltpu.VMEM((16 * _NB1,), jnp.int32),
        ],
    )(keys_flat)


def _sc_hist2_body(keys_hbm, b1_hbm, hist_hbm, buf, lh, bv):
    wid = _wid()
    zeros16 = jnp.zeros((16,), jnp.int32)

    def z(i, _):
        lh[pl.ds(i * 16, 16)] = zeros16
        return 0

    lax.fori_loop(0, 256, z, 0)
    pltpu.sync_copy(b1_hbm.at[0], bv)
    _load_chunk(keys_hbm, buf, wid)
    iota = lax.iota(jnp.int32, 16)
    ones = jnp.ones((16,), jnp.int32)
    vb1 = bv[pl.ds(0, 16)]

    def h(i, _):
        v = buf[pl.ds(i * 16, 16)]
        m = _bin1(v) == vb1
        idx = iota * 256 + ((v >> 5) & 0xFF)
        plsc.addupdate_scatter(lh, [idx], ones, mask=m)
        return 0

    lax.fori_loop(0, _VREGS, h, 0)

    def r(g, _):
        acc = lh[pl.ds(g * 16, 16)]
        for l in range(1, 16):
            acc = acc + lh[pl.ds(l * 256 + g * 16, 16)]
        buf[pl.ds(g * 16, 16)] = acc
        return 0

    lax.fori_loop(0, 16, r, 0)
    pltpu.sync_copy(buf.at[pl.ds(0, 256)], hist_hbm.at[wid])


def _sc_hist2(keys_flat, b1_vec):
    return pl.kernel(
        _sc_hist2_body,
        out_type=jax.ShapeDtypeStruct((_TILES, 256), jnp.int32),
        mesh=_sc_mesh(),
        compiler_params=pltpu.CompilerParams(needs_layout_passes=False),
        scratch_types=[
            pltpu.VMEM((_PERT,), jnp.int32),
            pltpu.VMEM((16 * 256,), jnp.int32),
            pltpu.VMEM((128,), jnp.int32),
        ],
    )(keys_flat, b1_vec)


def _sc_hist3_body(keys_hbm, b1_hbm, b2_hbm, hist_hbm, buf, lh, bv1, bv2):
    wid = _wid()
    zeros16 = jnp.zeros((16,), jnp.int32)

    def z(i, _):
        lh[pl.ds(i * 16, 16)] = zeros16
        return 0

    lax.fori_loop(0, 32, z, 0)
    pltpu.sync_copy(b1_hbm.at[0], bv1)
    pltpu.sync_copy(b2_hbm.at[0], bv2)
    _load_chunk(keys_hbm, buf, wid)
    iota = lax.iota(jnp.int32, 16)
    ones = jnp.ones((16,), jnp.int32)
    vb1 = bv1[pl.ds(0, 16)]
    vb2 = bv2[pl.ds(0, 16)]

    def h(i, _):
        v = buf[pl.ds(i * 16, 16)]
        m = (_bin1(v) == vb1) & (((v >> 5) & 0xFF) == vb2)
        idx = iota * 32 + (v & 0x1F)
        plsc.addupdate_scatter(lh, [idx], ones, mask=m)
        return 0

    lax.fori_loop(0, _VREGS, h, 0)

    def r(g, _):
        acc = lh[pl.ds(g * 16, 16)]
        for l in range(1, 16):
            acc = acc + lh[pl.ds(l * 32 + g * 16, 16)]
        buf[pl.ds(g * 16, 16)] = acc
        return 0

    lax.fori_loop(0, 2, r, 0)

    def zr(g, _):
        buf[pl.ds(32 + g * 16, 16)] = zeros16
        return 0

    lax.fori_loop(0, 6, zr, 0)
    pltpu.sync_copy(buf.at[pl.ds(0, 128)], hist_hbm.at[wid])


def _sc_hist3(keys_flat, b1_vec, b2_vec):
    return pl.kernel(
        _sc_hist3_body,
        out_type=jax.ShapeDtypeStruct((_TILES, 128), jnp.int32),
        mesh=_sc_mesh(),
        compiler_params=pltpu.CompilerParams(needs_layout_passes=False),
        scratch_types=[
            pltpu.VMEM((_PERT,), jnp.int32),
            pltpu.VMEM((16 * 32,), jnp.int32),
            pltpu.VMEM((128,), jnp.int32),
            pltpu.VMEM((128,), jnp.int32),
        ],
    )(keys_flat, b1_vec, b2_vec)


def _suffix_excl(g):
    R, L = g.shape
    src = lax.broadcasted_iota(jnp.int32, (L, L), 0)
    dst = lax.broadcasted_iota(jnp.int32, (L, L), 1)
    upper = (src > dst).astype(jnp.float32)
    w = lax.dot_general(g, upper, (((1,), (0,)), ((), ())),
                        preferred_element_type=jnp.float32)
    t = jnp.sum(g, axis=1, keepdims=True)
    rs = lax.broadcasted_iota(jnp.int32, (R, R), 0)
    rd = lax.broadcasted_iota(jnp.int32, (R, R), 1)
    later_rows = (rd > rs).astype(jnp.float32)
    carry = lax.dot_general(later_rows, t, (((1,), (0,)), ((), ())),
                            preferred_element_type=jnp.float32)
    return w + carry


def _crossing(g, suf, target):
    R, L = g.shape
    mask = (suf < target) & (suf + g >= target)
    bi = (lax.broadcasted_iota(jnp.int32, (R, L), 0) * L +
          lax.broadcasted_iota(jnp.int32, (R, L), 1))
    bstar = jnp.sum(jnp.where(mask, bi, 0))
    above = jnp.sum(jnp.where(mask, suf, 0.0)).astype(jnp.int32)
    return bstar, above


def _r1_body(h_ref, out_ref):
    h = h_ref[...].astype(jnp.float32)
    g = jnp.sum(h, axis=0)
    suf = _suffix_excl(g)
    bstar, a1 = _crossing(g, suf, float(_KPRE))
    lane = lax.broadcasted_iota(jnp.int32, (1, 128), 1)
    out_ref[...] = jnp.where(lane == 0, bstar, jnp.where(lane == 1, a1, 0))


def _r2_body(h_ref, r1_ref, out_ref):
    h = h_ref[...].astype(jnp.float32)
    a1 = r1_ref[0, 1]
    g = jnp.sum(h, axis=0)
    suf = _suffix_excl(g)
    b2, a2 = _crossing(g, suf, (_KPRE - a1).astype(jnp.float32))
    lane = lax.broadcasted_iota(jnp.int32, (1, 128), 1)
    out_ref[...] = jnp.where(lane == 0, b2, jnp.where(lane == 1, a2, 0))


def _r3_body(h1_ref, h2_ref, h3_ref, r1_ref, r2_ref,
             kstar_ref, o_ref, quota_ref, ts_ref):
    b1s = r1_ref[0, 0]
    a1 = r1_ref[0, 1]
    b2s = r2_ref[0, 0]
    a2 = r2_ref[0, 1]
    h3 = h3_ref[...].astype(jnp.float32)
    g3 = jnp.sum(h3, axis=0, keepdims=True)
    suf3 = _suffix_excl(g3)
    need2 = (_KPRE - a1 - a2).astype(jnp.float32)
    b3s, a3 = _crossing(g3, suf3, need2)
    a_total = a1 + a2 + a3
    tcnt = _KPRE - a_total

    key_normal = ((b1s - 1 + _BASE13) << 13) | (b2s << 5) | b3s
    key_bin0 = jnp.where(b2s == 0xFF, jnp.int32(-1065353217),
                         jnp.int32(_INT_MIN))
    kstar = jnp.where(b1s >= 1, key_normal, key_bin0)

    h1 = h1_ref[...].astype(jnp.float32)
    bi1 = (lax.broadcasted_iota(jnp.int32, (36, 128), 0) * 128 +
           lax.broadcasted_iota(jnp.int32, (36, 128), 1))
    g1_t = jnp.sum(h1 * (bi1 > b1s).astype(jnp.float32)[None], axis=(1, 2))
    h2 = h2_ref[...].astype(jnp.float32)
    bi2 = (lax.broadcasted_iota(jnp.int32, (2, 128), 0) * 128 +
           lax.broadcasted_iota(jnp.int32, (2, 128), 1))
    g2_t = jnp.sum(h2 * (bi2 > b2s).astype(jnp.float32)[None], axis=(1, 2))
    bi3 = lax.broadcasted_iota(jnp.int32, (_TILES, 128), 1)
    g3_t = jnp.sum(h3 * (bi3 > b3s).astype(jnp.float32), axis=1)
    c_t = jnp.sum(h3 * (bi3 == b3s).astype(jnp.float32), axis=1)
    g_t = (g1_t + g2_t + g3_t).reshape(1, _TILES)
    c_t = c_t.reshape(1, _TILES)

    ts_i = lax.broadcasted_iota(jnp.int32, (_TILES, _TILES), 0)
    td_i = lax.broadcasted_iota(jnp.int32, (_TILES, _TILES), 1)
    lower = (ts_i < td_i).astype(jnp.float32)
    p_t = lax.dot_general(c_t, lower, (((1,), (0,)), ((), ())),
                          preferred_element_type=jnp.float32)
    tf = tcnt.astype(jnp.float32)
    quota_t = jnp.clip(tf - p_t, 0.0, c_t)

    def c16(x):
        return jnp.floor((x + 15.0) * (1.0 / 16.0)) * 16.0

    g16 = c16(g_t)
    q16 = c16(quota_t)
    o_t = lax.dot_general(g16, lower, (((1,), (0,)), ((), ())),
                          preferred_element_type=jnp.float32)
    tie_base = jnp.sum(g16)
    ts_t = tie_base + lax.dot_general(q16, lower, (((1,), (0,)), ((), ())),
                                      preferred_element_type=jnp.float32)
    fill_start = tie_base + jnp.sum(q16)

    kstar_ref[...] = jnp.broadcast_to(kstar, (1, 128))
    lanes = lax.broadcasted_iota(jnp.int32, (1, 128), 1)
    zpad = jnp.zeros((1, 128 - _TILES), jnp.float32)
    o_full = jnp.concatenate([o_t, zpad], axis=1).astype(jnp.int32)
    o_ref[...] = jnp.where(lanes == _TILES, fill_start.astype(jnp.int32),
                           o_full)
    quota_ref[...] = jnp.concatenate([quota_t, zpad], axis=1).astype(jnp.int32)
    ts_ref[...] = jnp.concatenate([ts_t, zpad], axis=1).astype(jnp.int32)


def _reduce_kernels(h1, h2, h3):
    h1_3d = h1.reshape(_TILES, 36, 128)
    h2_3d = h2.reshape(_TILES, 2, 128)
    r1 = pl.pallas_call(
        _r1_body, out_shape=jax.ShapeDtypeStruct((1, 128), jnp.int32),
    )(h1_3d)
    r2 = pl.pallas_call(
        _r2_body, out_shape=jax.ShapeDtypeStruct((1, 128), jnp.int32),
    )(h2_3d, r1)
    kstar, o_t, quota_t, ts_t = pl.pallas_call(
        _r3_body,
        out_shape=[
            jax.ShapeDtypeStruct((1, 16), jnp.int32),
            jax.ShapeDtypeStruct((1, _TILES), jnp.int32),
            jax.ShapeDtypeStruct((1, _TILES), jnp.int32),
            jax.ShapeDtypeStruct((1, _TILES), jnp.int32),
        ],
    )(h1_3d, h2_3d, h3, r1, r2)
    return r1, r2, kstar, o_t, quota_t, ts_t


def _sc_compact_body(keys_hbm, kvec_hbm, o_hbm, q_hbm, ts_hbm,
                     outk_hbm, outi_hbm,
                     buf, gbk, gbi, tbi, tbk, kv16, sco, scq, scts):
    wid = _wid()
    base = wid * _PERT
    pltpu.sync_copy(kvec_hbm.at[0], kv16)
    pltpu.sync_copy(o_hbm.at[0], sco)
    pltpu.sync_copy(q_hbm.at[0], scq)
    pltpu.sync_copy(ts_hbm.at[0], scts)
    _load_chunk(keys_hbm, buf, wid)
    iota = lax.iota(jnp.int32, 16)
    kv = kv16[pl.ds(0, 16)]

    def scal(vec_ref, which):
        half = which // 16
        lane = which % 16
        vec = vec_ref[pl.ds(half * 16, 16)]
        return lax.reduce_max(jnp.where(iota == lane, vec, 0), axes=(0,))

    def step(i, carry):
        ng, nt = carry
        v = buf[pl.ds(i * 16, 16)]
        gm = v > kv
        tm = v == kv
        idxv = (base + i * 16) + iota
        plsc.store_compressed(gbk.at[pl.ds(ng, 16)], v, mask=gm)
        plsc.store_compressed(gbi.at[pl.ds(ng, 16)], idxv, mask=gm)
        tm2 = tm & (nt < 2032)
        plsc.store_compressed(tbi.at[pl.ds(nt, 16)], idxv, mask=tm2)
        cg = lax.reduce_max(plsc.all_reduce_population_count(gm), axes=(0,))
        ct = lax.reduce_max(plsc.all_reduce_population_count(tm2), axes=(0,))
        return ng + cg, nt + ct

    ng, nt = lax.fori_loop(0, _VREGS, step, (jnp.int32(0), jnp.int32(0)))

    my_o = scal(sco, wid)
    my_q = scal(scq, wid)
    my_ts = scal(scts, wid)
    mn16 = jnp.full((16,), _INT_MIN, jnp.int32)
    z16 = jnp.zeros((16,), jnp.int32)

    j0 = (ng // 16) * 16
    gk_tail = gbk[pl.ds(j0, 16)]
    gi_tail = gbi[pl.ds(j0, 16)]
    tailpos = j0 + iota
    gbk[pl.ds(j0, 16)] = jnp.where(tailpos < ng, gk_tail, mn16)
    gbi[pl.ds(j0, 16)] = jnp.where(tailpos < ng, gi_tail, z16)

    def cpg(j, _):
        off = pl.multiple_of(my_o + j * 16, 16)
        pltpu.sync_copy(gbk.at[pl.ds(j * 16, 16)],
                        outk_hbm.at[pl.ds(off, 16)])
        pltpu.sync_copy(gbi.at[pl.ds(j * 16, 16)],
                        outi_hbm.at[pl.ds(off, 16)])
        return 0

    lax.fori_loop(0, (ng + 15) // 16, cpg, 0)

    jq = (my_q // 16) * 16
    ti_tail = tbi[pl.ds(jq, 16)]
    qpos = jq + iota
    tbi[pl.ds(jq, 16)] = jnp.where(qpos < my_q, ti_tail, z16)

    def cpt(j, _):
        tpos = j * 16 + iota
        tbk[pl.ds(0, 16)] = jnp.where(tpos < my_q, kv, mn16)
        off = pl.multiple_of(my_ts + j * 16, 16)
        pltpu.sync_copy(tbk.at[pl.ds(0, 16)],
                        outk_hbm.at[pl.ds(off, 16)])
        pltpu.sync_copy(tbi.at[pl.ds(j * 16, 16)],
                        outi_hbm.at[pl.ds(off, 16)])
        return 0

    lax.fori_loop(0, (my_q + 15) // 16, cpt, 0)

    @pl.when(wid == _TILES - 1)
    def _():
        fill = scal(sco, _TILES)

        def fmn(j, _):
            tbk[pl.ds(j * 16, 16)] = mn16
            tbi[pl.ds(j * 16, 16)] = z16
            return 0

        lax.fori_loop(0, 32, fmn, 0)
        nbig = (4096 - fill) // 512

        def fbig(j, _):
            off = pl.multiple_of(fill + j * 512, 16)
            pltpu.sync_copy(tbk.at[pl.ds(0, 512)],
                            outk_hbm.at[pl.ds(off, 512)])
            pltpu.sync_copy(tbi.at[pl.ds(0, 512)],
                            outi_hbm.at[pl.ds(off, 512)])
            return 0

        lax.fori_loop(0, nbig, fbig, 0)
        start2 = fill + nbig * 512

        def fsm(j, _):
            off = pl.multiple_of(start2 + j * 16, 16)
            pltpu.sync_copy(tbk.at[pl.ds(0, 16)],
                            outk_hbm.at[pl.ds(off, 16)])
            pltpu.sync_copy(tbi.at[pl.ds(0, 16)],
                            outi_hbm.at[pl.ds(off, 16)])
            return 0

        lax.fori_loop(0, (4096 - start2) // 16, fsm, 0)


def _sc_compact(keys_flat, kstar_vec, o_t, quota_t, ts_t):
    return pl.kernel(
        _sc_compact_body,
        out_type=[
            jax.ShapeDtypeStruct((4096,), jnp.int32),
            jax.ShapeDtypeStruct((4096,), jnp.int32),
        ],
        mesh=_sc_mesh(),
        compiler_params=pltpu.CompilerParams(needs_layout_passes=False),
        scratch_types=[
            pltpu.VMEM((_PERT,), jnp.int32),
            pltpu.VMEM((2064,), jnp.int32),
            pltpu.VMEM((2064,), jnp.int32),
            pltpu.VMEM((2064,), jnp.int32),
            pltpu.VMEM((512,), jnp.int32),
            pltpu.VMEM((128,), jnp.int32),
            pltpu.VMEM((128,), jnp.int32),
            pltpu.VMEM((128,), jnp.int32),
            pltpu.VMEM((128,), jnp.int32),
        ],
    )(keys_flat, kstar_vec, o_t, quota_t, ts_t)


def _sort_body(k_ref, v_ref, ok_ref, ov_ref):
    keys = k_ref[...]
    vals = v_ref[...]
    i = lax.broadcasted_iota(jnp.int32, (1, 4096), 1)
    for size_p in range(1, 13):
        size = 1 << size_p
        dirm = (i & size) == 0
        for j_p in range(size_p - 1, -1, -1):
            j = 1 << j_p
            upperm = (i & j) != 0
            pk = jnp.where(upperm, pltpu.roll(keys, j, 1),
                           pltpu.roll(keys, 4096 - j, 1))
            pv = jnp.where(upperm, pltpu.roll(vals, j, 1),
                           pltpu.roll(vals, 4096 - j, 1))
            pf = (pk > keys) | ((pk == keys) & (pv < vals))
            take = pf ^ upperm ^ dirm ^ True
            keys = jnp.where(take, pk, keys)
            vals = jnp.where(take, pv, vals)
    ok_ref[...] = keys
    ov_ref[...] = vals


def _sort4096(keys, vals):
    return pl.pallas_call(
        _sort_body,
        out_shape=[jax.ShapeDtypeStruct((1, 4096), jnp.int32)] * 2,
    )(keys.reshape(1, 4096), vals.reshape(1, 4096))


_KP = 2048
_NB = 128
_NBLK = _KP // _NB


def _nms_body(x1c, y1c, x2c, y2c, x1r, y1r, x2r, y2r, keep_ref,
              iou_s, bb_s, kb_s):
    ax1 = x1r[...]
    ay1 = y1r[...]
    ax2 = x2r[...]
    ay2 = y2r[...]
    area_r = jnp.maximum(ax2 - ax1, 0.0) * jnp.maximum(ay2 - ay1, 0.0)
    keep_ref[...] = jnp.ones((1, _KP), jnp.float32)
    lane = lax.broadcasted_iota(jnp.int32, (1, _NB), 1)
    col = lax.broadcasted_iota(jnp.int32, (1, _KP), 1)

    for bi in range(_NBLK):
        s = bi * _NB
        xb1 = x1c[s:s + _NB, :]
        yb1 = y1c[s:s + _NB, :]
        xb2 = x2c[s:s + _NB, :]
        yb2 = y2c[s:s + _NB, :]
        area_b = jnp.maximum(xb2 - xb1, 0.0) * jnp.maximum(yb2 - yb1, 0.0)
        xx1 = jnp.maximum(xb1, ax1)
        yy1 = jnp.maximum(yb1, ay1)
        xx2 = jnp.minimum(xb2, ax2)
        yy2 = jnp.minimum(yb2, ay2)
        inter = jnp.maximum(xx2 - xx1, 0.0) * jnp.maximum(yy2 - yy1, 0.0)
        iou = inter / (area_b + area_r - inter + 1e-9)
        hit = (iou > _NMS_THRESH).astype(jnp.float32)
        iou_s[...] = hit
        bb_s[...] = hit[:, s:s + _NB]
        kb_s[...] = keep_ref[0:1, s:s + _NB]

        def intra(i, _):
            row = bb_s[pl.ds(i, 1), :]
            kb = kb_s[...]
            ki = jnp.max(jnp.where(lane == i, kb, 0.0))
            sup = (row > 0.0) & (lane > i) & (ki > 0.0)
            kb_s[...] = jnp.where(sup, 0.0, kb)
            return 0

        lax.fori_loop(0, _NB, intra, 0, unroll=False)

        kept = kb_s[...]
        keep_ref[0:1, s:s + _NB] = kept
        sup_all = lax.dot_general(
            kept, iou_s[...],
            dimension_numbers=(((1,), (0,)), ((), ())),
            preferred_element_type=jnp.float32)
        later = col >= (s + _NB)
        keep_ref[...] = jnp.where((sup_all > 0.0) & later, 0.0,
                                  keep_ref[...])


def _nms_keep_pallas(boxes):
    pad = _KP - _KPRE
    b = jnp.pad(boxes, ((0, pad), (0, 0)))
    cols = [b[:, i:i + 1] for i in range(4)]
    rows = [b[:, i].reshape(1, _KP) for i in range(4)]
    keep = pl.pallas_call(
        _nms_body,
        out_shape=jax.ShapeDtypeStruct((1, _KP), jnp.float32),
        scratch_shapes=[
            pltpu.VMEM((_NB, _KP), jnp.float32),
            pltpu.VMEM((_NB, _NB), jnp.float32),
            pltpu.VMEM((1, _NB), jnp.float32),
        ],
    )(*cols, *rows)
    return keep[0, :_KPRE] > 0.0


@jax.jit
def kernel(class_logits, box_regression, proposals):
    keys, bx1, by1, bx2, by2 = _dense_stage(
        class_logits, box_regression, proposals)
    keys_flat = keys.reshape(-1)

    h1 = _sc_hist1(keys_flat)
    r1 = pl.pallas_call(
        _r1_body, out_shape=jax.ShapeDtypeStruct((1, 128), jnp.int32),
    )(h1.reshape(_TILES, 36, 128))
    b1_vec = jnp.broadcast_to(r1[0:1, 0:1], (1, 128))
    h2 = _sc_hist2(keys_flat, b1_vec)
    r2 = pl.pallas_call(
        _r2_body, out_shape=jax.ShapeDtypeStruct((1, 128), jnp.int32),
    )(h2.reshape(_TILES, 2, 128), r1)
    b2_vec = jnp.broadcast_to(r2[0:1, 0:1], (1, 128))
    h3 = _sc_hist3(keys_flat, b1_vec, b2_vec)
    kstar, o_t, quota_t, ts_t = pl.pallas_call(
        _r3_body,
        out_shape=[jax.ShapeDtypeStruct((1, 128), jnp.int32)] * 4,
    )(h1.reshape(_TILES, 36, 128), h2.reshape(_TILES, 2, 128), h3, r1, r2)

    outk, outi = _sc_compact(keys_flat, kstar, o_t, quota_t, ts_t)
    sk, sv = _sort4096(outk, outi)

    top_idx = sv[0, :_KPRE]
    skk = sk[0, :_KPRE]
    vbits = jnp.where(skk >= 0, skk, skk ^ 0x7FFFFFFF)
    top_vals = lax.bitcast_convert_type(vbits, jnp.float32)

    sel_x1 = bx1.reshape(-1)[top_idx]
    sel_y1 = by1.reshape(-1)[top_idx]
    sel_x2 = bx2.reshape(-1)[top_idx]
    sel_y2 = by2.reshape(-1)[top_idx]
    sel_boxes = jnp.stack([sel_x1, sel_y1, sel_x2, sel_y2], axis=1)
    sel_labels = (top_idx % 90) + 1
    max_coord = jnp.max(sel_boxes)
    offsets = sel_labels.astype(jnp.float32) * (max_coord + 1.0)
    keep = _nms_keep_pallas(sel_boxes + offsets[:, None])
    final_scores = jnp.where(keep & (top_vals > _SCORE_THRESH), top_vals, -1.0)
    fvals, fidx = jax.lax.top_k(final_scores, _DET)
    out = jnp.concatenate([sel_boxes[fidx], fvals[:, None]], axis=1)
    return jnp.where((fvals > _SCORE_THRESH)[:, None], out, 0.0)


# speedup vs baseline: 37.6450x; 1.0050x over previous
import functools
import jax
import jax.numpy as jnp
import numpy as np
from jax import lax
from jax.experimental import pallas as pl
from jax.experimental.pallas import tpu as pltpu
from jax.experimental.pallas import tpu_sc as plsc

_N = 20000
_C = 91
_IMG_H = 800.0
_IMG_W = 1066.0
_SCORE_THRESH = 0.05
_NMS_THRESH = 0.5
_DET = 100
_KPRE = 2000
_CLIP = float(np.log(1000.0 / 16.0))

_BR = 400

_TOT = _N * 90
_TILES = 32
_PERT = 56256
_LAST = _TOT - 31 * _PERT
_VREGS = _PERT // 16
_INT_MIN = -2147483648
_BASE13 = 0x3D4CCCCE >> 13
_NB1 = 4608


def _dense_body(lg_ref, dx_ref, dy_ref, dw_ref, dh_ref, pr_ref,
                key_ref, bx1_ref, by1_ref, bx2_ref, by2_ref):
    logits = lg_ref[...]
    m = jnp.max(logits, axis=-1, keepdims=True)
    e = jnp.exp(logits - m)
    scores = (e / jnp.sum(e, axis=-1, keepdims=True))[:, 1:]

    px1 = pr_ref[:, 0]
    py1 = pr_ref[:, 1]
    px2 = pr_ref[:, 2]
    py2 = pr_ref[:, 3]
    widths = px2 - px1
    heights = py2 - py1
    ctr_x = px1 + 0.5 * widths
    ctr_y = py1 + 0.5 * heights

    dx = dx_ref[...] * (1.0 / 10.0)
    dy = dy_ref[...] * (1.0 / 10.0)
    dw = jnp.minimum(dw_ref[...] * (1.0 / 5.0), _CLIP)
    dh = jnp.minimum(dh_ref[...] * (1.0 / 5.0), _CLIP)

    pcx = dx * widths[:, None] + ctr_x[:, None]
    pcy = dy * heights[:, None] + ctr_y[:, None]
    pw = jnp.exp(dw) * widths[:, None]
    ph = jnp.exp(dh) * heights[:, None]

    bx1 = jnp.clip(pcx - 0.5 * pw, 0.0, _IMG_W)
    by1 = jnp.clip(pcy - 0.5 * ph, 0.0, _IMG_H)
    bx2 = jnp.clip(pcx + 0.5 * pw, 0.0, _IMG_W)
    by2 = jnp.clip(pcy + 0.5 * ph, 0.0, _IMG_H)

    ws = bx2 - bx1
    hs = by2 - by1
    valid = (scores > _SCORE_THRESH) & (ws >= 1e-2) & (hs >= 1e-2)
    masked = jnp.where(valid, scores, -1.0)
    v = lax.bitcast_convert_type(masked, jnp.int32)
    key_ref[...] = jnp.where(v >= 0, v, v ^ 0x7FFFFFFF)
    bx1_ref[...] = bx1
    by1_ref[...] = by1
    bx2_ref[...] = bx2
    by2_ref[...] = by2


def _dense_stage(class_logits, box_regression, proposals):
    reg = box_regression
    dx = reg[:, 4::4]
    dy = reg[:, 5::4]
    dw = reg[:, 6::4]
    dh = reg[:, 7::4]

    grid = (_N // _BR,)
    row_spec = pl.BlockSpec((_BR, 90), lambda i: (i, 0))
    fshape = jax.ShapeDtypeStruct((_N, 90), jnp.float32)
    ishape = jax.ShapeDtypeStruct((_N, 90), jnp.int32)
    return pl.pallas_call(
        _dense_body,
        grid=grid,
        in_specs=[
            pl.BlockSpec((_BR, _C), lambda i: (i, 0)),
            row_spec, row_spec, row_spec, row_spec,
            pl.BlockSpec((_BR, 4), lambda i: (i, 0)),
        ],
        out_specs=[row_spec] * 5,
        out_shape=[ishape, fshape, fshape, fshape, fshape],
    )(class_logits, dx, dy, dw, dh, proposals)


def _sc_mesh():
    return plsc.VectorSubcoreMesh(core_axis_name="c", subcore_axis_name="s")


def _wid():
    return lax.axis_index("s") * 2 + lax.axis_index("c")


def _load_chunk(keys_hbm, buf, wid):
    base = wid * _PERT

    @pl.when(wid == _TILES - 1)
    def _():
        mn = jnp.full((16,), _INT_MIN, jnp.int32)

        def zt(i, _):
            buf[pl.ds(_LAST + i * 16, 16)] = mn
            return 0

        lax.fori_loop(0, (_PERT - _LAST) // 16, zt, 0)
        pltpu.sync_copy(keys_hbm.at[pl.ds(base, _LAST)],
                        buf.at[pl.ds(0, _LAST)])

    @pl.when(wid < _TILES - 1)
    def _():
        pltpu.sync_copy(keys_hbm.at[pl.ds(base, _PERT)], buf)


def _bin1(v):
    return jnp.clip((v >> 13) - (_BASE13 - 1), 0, _NB1 - 1)


def _sc_hist1_body(keys_hbm, hist_hbm, buf, lh):
    wid = _wid()
    zeros16 = jnp.zeros((16,), jnp.int32)

    def z(i, _):
        lh[pl.ds(i * 16, 16)] = zeros16
        return 0

    lax.fori_loop(0, _NB1, z, 0)
    _load_chunk(keys_hbm, buf, wid)
    iota = lax.iota(jnp.int32, 16)
    ones = jnp.ones((16,), jnp.int32)

    def h(i, _):
        v = buf[pl.ds(i * 16, 16)]
        idx = iota * _NB1 + _bin1(v)
        plsc.addupdate_scatter(lh, [idx], ones)
        return 0

    lax.fori_loop(0, _VREGS, h, 0)

    def r(g, _):
        acc = lh[pl.ds(g * 16, 16)]
        for l in range(1, 16):
            acc = acc + lh[pl.ds(l * _NB1 + g * 16, 16)]
        buf[pl.ds(g * 16, 16)] = acc
        return 0

    lax.fori_loop(0, _NB1 // 16, r, 0)
    pltpu.sync_copy(buf.at[pl.ds(0, _NB1)], hist_hbm.at[wid])


def _sc_hist1(keys_flat):
    return pl.kernel(
        _sc_hist1_body,
        out_type=jax.ShapeDtypeStruct((_TILES, _NB1), jnp.int32),
        mesh=_sc_mesh(),
        compiler_params=pltpu.CompilerParams(needs_layout_passes=False),
        scratch_types=[
            pltpu.VMEM((_PERT,), jnp.int32),
            pltpu.VMEM((16 * _NB1,), jnp.int32),
        ],
    )(keys_flat)


def _sc_hist2_body(keys_hbm, b1_hbm, hist_hbm, buf, lh, bv):
    wid = _wid()
    zeros16 = jnp.zeros((16,), jnp.int32)

    def z(i, _):
        lh[pl.ds(i * 16, 16)] = zeros16
        return 0

    lax.fori_loop(0, 256, z, 0)
    pltpu.sync_copy(b1_hbm.at[0], bv)
    _load_chunk(keys_hbm, buf, wid)
    iota = lax.iota(jnp.int32, 16)
    ones = jnp.ones((16,), jnp.int32)
    vb1 = bv[pl.ds(0, 16)]

    def h(i, _):
        v = buf[pl.ds(i * 16, 16)]
        m = _bin1(v) == vb1
        idx = iota * 256 + ((v >> 5) & 0xFF)
        plsc.addupdate_scatter(lh, [idx], ones, mask=m)
        return 0

    lax.fori_loop(0, _VREGS, h, 0)

    def r(g, _):
        acc = lh[pl.ds(g * 16, 16)]
        for l in range(1, 16):
            acc = acc + lh[pl.ds(l * 256 + g * 16, 16)]
        buf[pl.ds(g * 16, 16)] = acc
        return 0

    lax.fori_loop(0, 16, r, 0)
    pltpu.sync_copy(buf.at[pl.ds(0, 256)], hist_hbm.at[wid])


def _sc_hist2(keys_flat, b1_vec):
    return pl.kernel(
        _sc_hist2_body,
        out_type=jax.ShapeDtypeStruct((_TILES, 256), jnp.int32),
        mesh=_sc_mesh(),
        compiler_params=pltpu.CompilerParams(needs_layout_passes=False),
        scratch_types=[
            pltpu.VMEM((_PERT,), jnp.int32),
            pltpu.VMEM((16 * 256,), jnp.int32),
            pltpu.VMEM((128,), jnp.int32),
        ],
    )(keys_flat, b1_vec)


def _sc_hist3_body(keys_hbm, b1_hbm, b2_hbm, hist_hbm, buf, lh, bv1, bv2):
    wid = _wid()
    zeros16 = jnp.zeros((16,), jnp.int32)

    def z(i, _):
        lh[pl.ds(i * 16, 16)] = zeros16
        return 0

    lax.fori_loop(0, 32, z, 0)
    pltpu.sync_copy(b1_hbm.at[0], bv1)
    pltpu.sync_copy(b2_hbm.at[0], bv2)
    _load_chunk(keys_hbm, buf, wid)
    iota = lax.iota(jnp.int32, 16)
    ones = jnp.ones((16,), jnp.int32)
    vb1 = bv1[pl.ds(0, 16)]
    vb2 = bv2[pl.ds(0, 16)]

    def h(i, _):
        v = buf[pl.ds(i * 16, 16)]
        m = (_bin1(v) == vb1) & (((v >> 5) & 0xFF) == vb2)
        idx = iota * 32 + (v & 0x1F)
        plsc.addupdate_scatter(lh, [idx], ones, mask=m)
        return 0

    lax.fori_loop(0, _VREGS, h, 0)

    def r(g, _):
        acc = lh[pl.ds(g * 16, 16)]
        for l in range(1, 16):
            acc = acc + lh[pl.ds(l * 32 + g * 16, 16)]
        buf[pl.ds(g * 16, 16)] = acc
        return 0

    lax.fori_loop(0, 2, r, 0)

    def zr(g, _):
        buf[pl.ds(32 + g * 16, 16)] = zeros16
        return 0

    lax.fori_loop(0, 6, zr, 0)
    pltpu.sync_copy(buf.at[pl.ds(0, 128)], hist_hbm.at[wid])


def _sc_hist3(keys_flat, b1_vec, b2_vec):
    return pl.kernel(
        _sc_hist3_body,
        out_type=jax.ShapeDtypeStruct((_TILES, 128), jnp.int32),
        mesh=_sc_mesh(),
        compiler_params=pltpu.CompilerParams(needs_layout_passes=False),
        scratch_types=[
            pltpu.VMEM((_PERT,), jnp.int32),
            pltpu.VMEM((16 * 32,), jnp.int32),
            pltpu.VMEM((128,), jnp.int32),
            pltpu.VMEM((128,), jnp.int32),
        ],
    )(keys_flat, b1_vec, b2_vec)


def _suffix_excl(g):
    R, L = g.shape
    src = lax.broadcasted_iota(jnp.int32, (L, L), 0)
    dst = lax.broadcasted_iota(jnp.int32, (L, L), 1)
    upper = (src > dst).astype(jnp.float32)
    w = lax.dot_general(g, upper, (((1,), (0,)), ((), ())),
                        preferred_element_type=jnp.float32)
    t = jnp.sum(g, axis=1, keepdims=True)
    rs = lax.broadcasted_iota(jnp.int32, (R, R), 0)
    rd = lax.broadcasted_iota(jnp.int32, (R, R), 1)
    later_rows = (rd > rs).astype(jnp.float32)
    carry = lax.dot_general(later_rows, t, (((1,), (0,)), ((), ())),
                            preferred_element_type=jnp.float32)
    return w + carry


def _crossing(g, suf, target):
    R, L = g.shape
    mask = (suf < target) & (suf + g >= target)
    bi = (lax.broadcasted_iota(jnp.int32, (R, L), 0) * L +
          lax.broadcasted_iota(jnp.int32, (R, L), 1))
    bstar = jnp.sum(jnp.where(mask, bi, 0))
    above = jnp.sum(jnp.where(mask, suf, 0.0)).astype(jnp.int32)
    return bstar, above


def _r1_body(h_ref, out_ref):
    h = h_ref[...].astype(jnp.float32)
    g = jnp.sum(h, axis=0)
    suf = _suffix_excl(g)
    bstar, a1 = _crossing(g, suf, float(_KPRE))
    lane = lax.broadcasted_iota(jnp.int32, (1, 128), 1)
    out_ref[...] = jnp.where(lane == 0, bstar, jnp.where(lane == 1, a1, 0))


def _r2_body(h_ref, r1_ref, out_ref):
    h = h_ref[...].astype(jnp.float32)
    a1 = r1_ref[0, 1]
    g = jnp.sum(h, axis=0)
    suf = _suffix_excl(g)
    b2, a2 = _crossing(g, suf, (_KPRE - a1).astype(jnp.float32))
    lane = lax.broadcasted_iota(jnp.int32, (1, 128), 1)
    out_ref[...] = jnp.where(lane == 0, b2, jnp.where(lane == 1, a2, 0))


def _r3_body(h1_ref, h2_ref, h3_ref, r1_ref, r2_ref,
             kstar_ref, o_ref, quota_ref, ts_ref):
    b1s = r1_ref[0, 0]
    a1 = r1_ref[0, 1]
    b2s = r2_ref[0, 0]
    a2 = r2_ref[0, 1]
    h3 = h3_ref[...].astype(jnp.float32)
    g3 = jnp.sum(h3, axis=0, keepdims=True)
    suf3 = _suffix_excl(g3)
    need2 = (_KPRE - a1 - a2).astype(jnp.float32)
    b3s, a3 = _crossing(g3, suf3, need2)
    a_total = a1 + a2 + a3
    tcnt = _KPRE - a_total

    key_normal = ((b1s - 1 + _BASE13) << 13) | (b2s << 5) | b3s
    key_bin0 = jnp.where(b2s == 0xFF, jnp.int32(-1065353217),
                         jnp.int32(_INT_MIN))
    kstar = jnp.where(b1s >= 1, key_normal, key_bin0)

    h1 = h1_ref[...].astype(jnp.float32)
    bi1 = (lax.broadcasted_iota(jnp.int32, (36, 128), 0) * 128 +
           lax.broadcasted_iota(jnp.int32, (36, 128), 1))
    g1_t = jnp.sum(h1 * (bi1 > b1s).astype(jnp.float32)[None], axis=(1, 2))
    h2 = h2_ref[...].astype(jnp.float32)
    bi2 = (lax.broadcasted_iota(jnp.int32, (2, 128), 0) * 128 +
           lax.broadcasted_iota(jnp.int32, (2, 128), 1))
    g2_t = jnp.sum(h2 * (bi2 > b2s).astype(jnp.float32)[None], axis=(1, 2))
    bi3 = lax.broadcasted_iota(jnp.int32, (_TILES, 128), 1)
    g3_t = jnp.sum(h3 * (bi3 > b3s).astype(jnp.float32), axis=1)
    c_t = jnp.sum(h3 * (bi3 == b3s).astype(jnp.float32), axis=1)
    g_t = (g1_t + g2_t + g3_t).reshape(1, _TILES)
    c_t = c_t.reshape(1, _TILES)

    ts_i = lax.broadcasted_iota(jnp.int32, (_TILES, _TILES), 0)
    td_i = lax.broadcasted_iota(jnp.int32, (_TILES, _TILES), 1)
    lower = (ts_i < td_i).astype(jnp.float32)
    p_t = lax.dot_general(c_t, lower, (((1,), (0,)), ((), ())),
                          preferred_element_type=jnp.float32)
    tf = tcnt.astype(jnp.float32)
    quota_t = jnp.clip(tf - p_t, 0.0, c_t)

    def c16(x):
        return jnp.floor((x + 15.0) * (1.0 / 16.0)) * 16.0

    g16 = c16(g_t)
    q16 = c16(quota_t)
    o_t = lax.dot_general(g16, lower, (((1,), (0,)), ((), ())),
                          preferred_element_type=jnp.float32)
    tie_base = jnp.sum(g16)
    ts_t = tie_base + lax.dot_general(q16, lower, (((1,), (0,)), ((), ())),
                                      preferred_element_type=jnp.float32)
    fill_start = tie_base + jnp.sum(q16)

    kstar_ref[...] = jnp.broadcast_to(kstar, (1, 128))
    lanes = lax.broadcasted_iota(jnp.int32, (1, 128), 1)
    zpad = jnp.zeros((1, 128 - _TILES), jnp.float32)
    o_full = jnp.concatenate([o_t, zpad], axis=1).astype(jnp.int32)
    o_ref[...] = jnp.where(lanes == _TILES, fill_start.astype(jnp.int32),
                           o_full)
    quota_ref[...] = jnp.concatenate([quota_t, zpad], axis=1).astype(jnp.int32)
    ts_ref[...] = jnp.concatenate([ts_t, zpad], axis=1).astype(jnp.int32)


def _reduce_kernels(h1, h2, h3):
    h1_3d = h1.reshape(_TILES, 36, 128)
    h2_3d = h2.reshape(_TILES, 2, 128)
    r1 = pl.pallas_call(
        _r1_body, out_shape=jax.ShapeDtypeStruct((1, 128), jnp.int32),
    )(h1_3d)
    r2 = pl.pallas_call(
        _r2_body, out_shape=jax.ShapeDtypeStruct((1, 128), jnp.int32),
    )(h2_3d, r1)
    kstar, o_t, quota_t, ts_t = pl.pallas_call(
        _r3_body,
        out_shape=[
            jax.ShapeDtypeStruct((1, 16), jnp.int32),
            jax.ShapeDtypeStruct((1, _TILES), jnp.int32),
            jax.ShapeDtypeStruct((1, _TILES), jnp.int32),
            jax.ShapeDtypeStruct((1, _TILES), jnp.int32),
        ],
    )(h1_3d, h2_3d, h3, r1, r2)
    return r1, r2, kstar, o_t, quota_t, ts_t


def _sc_compact_body(keys_hbm, kvec_hbm, o_hbm, q_hbm, ts_hbm,
                     outk_hbm, outi_hbm,
                     buf, gbk, gbi, tbi, tbk, kv16, sco, scq, scts):
    wid = _wid()
    base = wid * _PERT
    pltpu.sync_copy(kvec_hbm.at[0], kv16)
    pltpu.sync_copy(o_hbm.at[0], sco)
    pltpu.sync_copy(q_hbm.at[0], scq)
    pltpu.sync_copy(ts_hbm.at[0], scts)
    _load_chunk(keys_hbm, buf, wid)
    iota = lax.iota(jnp.int32, 16)
    kv = kv16[pl.ds(0, 16)]

    def scal(vec_ref, which):
        half = which // 16
        lane = which % 16
        vec = vec_ref[pl.ds(half * 16, 16)]
        return lax.reduce_max(jnp.where(iota == lane, vec, 0), axes=(0,))

    def step(i, carry):
        ng, nt = carry
        v = buf[pl.ds(i * 16, 16)]
        gm = v > kv
        tm = v == kv
        idxv = (base + i * 16) + iota
        plsc.store_compressed(gbk.at[pl.ds(ng, 16)], v, mask=gm)
        plsc.store_compressed(gbi.at[pl.ds(ng, 16)], idxv, mask=gm)
        tm2 = tm & (nt < 2032)
        plsc.store_compressed(tbi.at[pl.ds(nt, 16)], idxv, mask=tm2)
        cg = lax.reduce_max(plsc.all_reduce_population_count(gm), axes=(0,))
        ct = lax.reduce_max(plsc.all_reduce_population_count(tm2), axes=(0,))
        return ng + cg, nt + ct

    ng, nt = lax.fori_loop(0, _VREGS, step, (jnp.int32(0), jnp.int32(0)))

    my_o = scal(sco, wid)
    my_q = scal(scq, wid)
    my_ts = scal(scts, wid)
    mn16 = jnp.full((16,), _INT_MIN, jnp.int32)
    z16 = jnp.zeros((16,), jnp.int32)

    j0 = (ng // 16) * 16
    gk_tail = gbk[pl.ds(j0, 16)]
    gi_tail = gbi[pl.ds(j0, 16)]
    tailpos = j0 + iota
    gbk[pl.ds(j0, 16)] = jnp.where(tailpos < ng, gk_tail, mn16)
    gbi[pl.ds(j0, 16)] = jnp.where(tailpos < ng, gi_tail, z16)

    def cpg(j, _):
        off = pl.multiple_of(my_o + j * 16, 16)
        pltpu.sync_copy(gbk.at[pl.ds(j * 16, 16)],
                        outk_hbm.at[pl.ds(off, 16)])
        pltpu.sync_copy(gbi.at[pl.ds(j * 16, 16)],
                        outi_hbm.at[pl.ds(off, 16)])
        return 0

    lax.fori_loop(0, (ng + 15) // 16, cpg, 0)

    jq = (my_q // 16) * 16
    ti_tail = tbi[pl.ds(jq, 16)]
    qpos = jq + iota
    tbi[pl.ds(jq, 16)] = jnp.where(qpos < my_q, ti_tail, z16)

    def cpt(j, _):
        tpos = j * 16 + iota
        tbk[pl.ds(0, 16)] = jnp.where(tpos < my_q, kv, mn16)
        off = pl.multiple_of(my_ts + j * 16, 16)
        pltpu.sync_copy(tbk.at[pl.ds(0, 16)],
                        outk_hbm.at[pl.ds(off, 16)])
        pltpu.sync_copy(tbi.at[pl.ds(j * 16, 16)],
                        outi_hbm.at[pl.ds(off, 16)])
        return 0

    lax.fori_loop(0, (my_q + 15) // 16, cpt, 0)

    @pl.when(wid == _TILES - 1)
    def _():
        fill = scal(sco, _TILES)

        def fmn(j, _):
            tbk[pl.ds(j * 16, 16)] = mn16
            tbi[pl.ds(j * 16, 16)] = z16
            return 0

        lax.fori_loop(0, 32, fmn, 0)
        nbig = (4096 - fill) // 512

        def fbig(j, _):
            off = pl.multiple_of(fill + j * 512, 16)
            pltpu.sync_copy(tbk.at[pl.ds(0, 512)],
                            outk_hbm.at[pl.ds(off, 512)])
            pltpu.sync_copy(tbi.at[pl.ds(0, 512)],
                            outi_hbm.at[pl.ds(off, 512)])
            return 0

        lax.fori_loop(0, nbig, fbig, 0)
        start2 = fill + nbig * 512

        def fsm(j, _):
            off = pl.multiple_of(start2 + j * 16, 16)
            pltpu.sync_copy(tbk.at[pl.ds(0, 16)],
                            outk_hbm.at[pl.ds(off, 16)])
            pltpu.sync_copy(tbi.at[pl.ds(0, 16)],
                            outi_hbm.at[pl.ds(off, 16)])
            return 0

        lax.fori_loop(0, (4096 - start2) // 16, fsm, 0)


def _sc_compact(keys_flat, kstar_vec, o_t, quota_t, ts_t):
    return pl.kernel(
        _sc_compact_body,
        out_type=[
            jax.ShapeDtypeStruct((4096,), jnp.int32),
            jax.ShapeDtypeStruct((4096,), jnp.int32),
        ],
        mesh=_sc_mesh(),
        compiler_params=pltpu.CompilerParams(needs_layout_passes=False),
        scratch_types=[
            pltpu.VMEM((_PERT,), jnp.int32),
            pltpu.VMEM((2064,), jnp.int32),
            pltpu.VMEM((2064,), jnp.int32),
            pltpu.VMEM((2064,), jnp.int32),
            pltpu.VMEM((512,), jnp.int32),
            pltpu.VMEM((128,), jnp.int32),
            pltpu.VMEM((128,), jnp.int32),
            pltpu.VMEM((128,), jnp.int32),
            pltpu.VMEM((128,), jnp.int32),
        ],
    )(keys_flat, kstar_vec, o_t, quota_t, ts_t)


def _sort_body(k_ref, v_ref, ok_ref, ov_ref):
    keys = k_ref[...]
    vals = v_ref[...]
    i = lax.broadcasted_iota(jnp.int32, (1, 4096), 1)
    for size_p in range(1, 13):
        size = 1 << size_p
        dirm = (i & size) == 0
        for j_p in range(size_p - 1, -1, -1):
            j = 1 << j_p
            upperm = (i & j) != 0
            pk = jnp.where(upperm, pltpu.roll(keys, j, 1),
                           pltpu.roll(keys, 4096 - j, 1))
            pv = jnp.where(upperm, pltpu.roll(vals, j, 1),
                           pltpu.roll(vals, 4096 - j, 1))
            pf = (pk > keys) | ((pk == keys) & (pv < vals))
            take = pf ^ upperm ^ dirm ^ True
            keys = jnp.where(take, pk, keys)
            vals = jnp.where(take, pv, vals)
    ok_ref[...] = keys
    ov_ref[...] = vals


def _sort4096(keys, vals):
    return pl.pallas_call(
        _sort_body,
        out_shape=[jax.ShapeDtypeStruct((1, 4096), jnp.int32)] * 2,
    )(keys.reshape(1, 4096), vals.reshape(1, 4096))


_KP = 2048
_NB = 128
_NBLK = _KP // _NB


def _nms_body(x1c, y1c, x2c, y2c, x1r, y1r, x2r, y2r, keep_ref,
              iou_s, bb_s, kb_s):
    ax1 = x1r[...]
    ay1 = y1r[...]
    ax2 = x2r[...]
    ay2 = y2r[...]
    area_r = jnp.maximum(ax2 - ax1, 0.0) * jnp.maximum(ay2 - ay1, 0.0)
    keep_ref[...] = jnp.ones((1, _KP), jnp.float32)
    lane = lax.broadcasted_iota(jnp.int32, (1, _NB), 1)
    col = lax.broadcasted_iota(jnp.int32, (1, _KP), 1)

    for bi in range(_NBLK):
        s = bi * _NB
        w = _KP - s
        xb1 = x1c[s:s + _NB, :]
        yb1 = y1c[s:s + _NB, :]
        xb2 = x2c[s:s + _NB, :]
        yb2 = y2c[s:s + _NB, :]
        area_b = jnp.maximum(xb2 - xb1, 0.0) * jnp.maximum(yb2 - yb1, 0.0)
        xx1 = jnp.maximum(xb1, ax1[:, s:])
        yy1 = jnp.maximum(yb1, ay1[:, s:])
        xx2 = jnp.minimum(xb2, ax2[:, s:])
        yy2 = jnp.minimum(yb2, ay2[:, s:])
        inter = jnp.maximum(xx2 - xx1, 0.0) * jnp.maximum(yy2 - yy1, 0.0)
        iou = inter / (area_b + area_r[:, s:] - inter + 1e-9)
        hit = (iou > _NMS_THRESH).astype(jnp.float32)
        iou_s[:, 0:w] = hit
        bb_s[...] = hit[:, 0:_NB]
        kb_s[...] = keep_ref[0:1, s:s + _NB]

        def intra(i, _):
            row = bb_s[pl.ds(i, 1), :]
            kb = kb_s[...]
            ki = jnp.max(jnp.where(lane == i, kb, 0.0))
            sup = (row > 0.0) & (lane > i) & (ki > 0.0)
            kb_s[...] = jnp.where(sup, 0.0, kb)
            return 0

        lax.fori_loop(0, _NB, intra, 0, unroll=False)

        kept = kb_s[...]
        keep_ref[0:1, s:s + _NB] = kept
        if bi < _NBLK - 1:
            sup_all = lax.dot_general(
                kept, iou_s[:, _NB:w],
                dimension_numbers=(((1,), (0,)), ((), ())),
                preferred_element_type=jnp.float32)
            kl = keep_ref[0:1, s + _NB:]
            keep_ref[0:1, s + _NB:] = jnp.where(sup_all > 0.0, 0.0, kl)


def _nms_keep_pallas(boxes):
    pad = _KP - _KPRE
    b = jnp.pad(boxes, ((0, pad), (0, 0)))
    cols = [b[:, i:i + 1] for i in range(4)]
    rows = [b[:, i].reshape(1, _KP) for i in range(4)]
    keep = pl.pallas_call(
        _nms_body,
        out_shape=jax.ShapeDtypeStruct((1, _KP), jnp.float32),
        scratch_shapes=[
            pltpu.VMEM((_NB, _KP), jnp.float32),
            pltpu.VMEM((_NB, _NB), jnp.float32),
            pltpu.VMEM((1, _NB), jnp.float32),
        ],
    )(*cols, *rows)
    return keep[0, :_KPRE] > 0.0


@jax.jit
def kernel(class_logits, box_regression, proposals):
    keys, bx1, by1, bx2, by2 = _dense_stage(
        class_logits, box_regression, proposals)
    keys_flat = keys.reshape(-1)

    h1 = _sc_hist1(keys_flat)
    r1 = pl.pallas_call(
        _r1_body, out_shape=jax.ShapeDtypeStruct((1, 128), jnp.int32),
    )(h1.reshape(_TILES, 36, 128))
    b1_vec = jnp.broadcast_to(r1[0:1, 0:1], (1, 128))
    h2 = _sc_hist2(keys_flat, b1_vec)
    r2 = pl.pallas_call(
        _r2_body, out_shape=jax.ShapeDtypeStruct((1, 128), jnp.int32),
    )(h2.reshape(_TILES, 2, 128), r1)
    b2_vec = jnp.broadcast_to(r2[0:1, 0:1], (1, 128))
    h3 = _sc_hist3(keys_flat, b1_vec, b2_vec)
    kstar, o_t, quota_t, ts_t = pl.pallas_call(
        _r3_body,
        out_shape=[jax.ShapeDtypeStruct((1, 128), jnp.int32)] * 4,
    )(h1.reshape(_TILES, 36, 128), h2.reshape(_TILES, 2, 128), h3, r1, r2)

    outk, outi = _sc_compact(keys_flat, kstar, o_t, quota_t, ts_t)
    sk, sv = _sort4096(outk, outi)

    top_idx = sv[0, :_KPRE]
    skk = sk[0, :_KPRE]
    vbits = jnp.where(skk >= 0, skk, skk ^ 0x7FFFFFFF)
    top_vals = lax.bitcast_convert_type(vbits, jnp.float32)

    sel_x1 = bx1.reshape(-1)[top_idx]
    sel_y1 = by1.reshape(-1)[top_idx]
    sel_x2 = bx2.reshape(-1)[top_idx]
    sel_y2 = by2.reshape(-1)[top_idx]
    sel_boxes = jnp.stack([sel_x1, sel_y1, sel_x2, sel_y2], axis=1)
    sel_labels = (top_idx % 90) + 1
    max_coord = jnp.max(sel_boxes)
    offsets = sel_labels.astype(jnp.float32) * (max_coord + 1.0)
    keep = _nms_keep_pallas(sel_boxes + offsets[:, None])
    final_scores = jnp.where(keep & (top_vals > _SCORE_THRESH), top_vals, -1.0)
    fvals, fidx = jax.lax.top_k(final_scores, _DET)
    out = jnp.concatenate([sel_boxes[fidx], fvals[:, None]], axis=1)
    return jnp.where((fvals > _SCORE_THRESH)[:, None], out, 0.0)


# speedup vs baseline: 37.7709x; 1.0033x over previous
import functools
import jax
import jax.numpy as jnp
import numpy as np
from jax import lax
from jax.experimental import pallas as pl
from jax.experimental.pallas import tpu as pltpu
from jax.experimental.pallas import tpu_sc as plsc

_N = 20000
_C = 91
_IMG_H = 800.0
_IMG_W = 1066.0
_SCORE_THRESH = 0.05
_NMS_THRESH = 0.5
_DET = 100
_KPRE = 2000
_CLIP = float(np.log(1000.0 / 16.0))

_BR = 400

_TOT = _N * 90
_TILES = 32
_PERT = 56256
_LAST = _TOT - 31 * _PERT
_VREGS = _PERT // 16
_INT_MIN = -2147483648
_BASE13 = 0x3D4CCCCE >> 13
_NB1 = 4608


def _dense_body(lg_ref, dx_ref, dy_ref, dw_ref, dh_ref, pr_ref,
                key_ref, bx1_ref, by1_ref, bx2_ref, by2_ref):
    logits = lg_ref[...]
    m = jnp.max(logits, axis=-1, keepdims=True)
    e = jnp.exp(logits - m)
    scores = (e / jnp.sum(e, axis=-1, keepdims=True))[:, 1:]

    px1 = pr_ref[:, 0]
    py1 = pr_ref[:, 1]
    px2 = pr_ref[:, 2]
    py2 = pr_ref[:, 3]
    widths = px2 - px1
    heights = py2 - py1
    ctr_x = px1 + 0.5 * widths
    ctr_y = py1 + 0.5 * heights

    dx = dx_ref[...] * (1.0 / 10.0)
    dy = dy_ref[...] * (1.0 / 10.0)
    dw = jnp.minimum(dw_ref[...] * (1.0 / 5.0), _CLIP)
    dh = jnp.minimum(dh_ref[...] * (1.0 / 5.0), _CLIP)

    pcx = dx * widths[:, None] + ctr_x[:, None]
    pcy = dy * heights[:, None] + ctr_y[:, None]
    pw = jnp.exp(dw) * widths[:, None]
    ph = jnp.exp(dh) * heights[:, None]

    bx1 = jnp.clip(pcx - 0.5 * pw, 0.0, _IMG_W)
    by1 = jnp.clip(pcy - 0.5 * ph, 0.0, _IMG_H)
    bx2 = jnp.clip(pcx + 0.5 * pw, 0.0, _IMG_W)
    by2 = jnp.clip(pcy + 0.5 * ph, 0.0, _IMG_H)

    ws = bx2 - bx1
    hs = by2 - by1
    valid = (scores > _SCORE_THRESH) & (ws >= 1e-2) & (hs >= 1e-2)
    masked = jnp.where(valid, scores, -1.0)
    v = lax.bitcast_convert_type(masked, jnp.int32)
    key_ref[...] = jnp.where(v >= 0, v, v ^ 0x7FFFFFFF)
    bx1_ref[...] = bx1
    by1_ref[...] = by1
    bx2_ref[...] = bx2
    by2_ref[...] = by2


def _dense_stage(class_logits, box_regression, proposals):
    reg = box_regression
    dx = reg[:, 4::4]
    dy = reg[:, 5::4]
    dw = reg[:, 6::4]
    dh = reg[:, 7::4]

    grid = (_N // _BR,)
    row_spec = pl.BlockSpec((_BR, 90), lambda i: (i, 0))
    fshape = jax.ShapeDtypeStruct((_N, 90), jnp.float32)
    ishape = jax.ShapeDtypeStruct((_N, 90), jnp.int32)
    return pl.pallas_call(
        _dense_body,
        grid=grid,
        in_specs=[
            pl.BlockSpec((_BR, _C), lambda i: (i, 0)),
            row_spec, row_spec, row_spec, row_spec,
            pl.BlockSpec((_BR, 4), lambda i: (i, 0)),
        ],
        out_specs=[row_spec] * 5,
        out_shape=[ishape, fshape, fshape, fshape, fshape],
    )(class_logits, dx, dy, dw, dh, proposals)


def _sc_mesh():
    return plsc.VectorSubcoreMesh(core_axis_name="c", subcore_axis_name="s")


def _wid():
    return lax.axis_index("s") * 2 + lax.axis_index("c")


def _load_chunk(keys_hbm, buf, wid):
    base = wid * _PERT

    @pl.when(wid == _TILES - 1)
    def _():
        mn = jnp.full((16,), _INT_MIN, jnp.int32)

        def zt(i, _):
            buf[pl.ds(_LAST + i * 16, 16)] = mn
            return 0

        lax.fori_loop(0, (_PERT - _LAST) // 16, zt, 0)
        pltpu.sync_copy(keys_hbm.at[pl.ds(base, _LAST)],
                        buf.at[pl.ds(0, _LAST)])

    @pl.when(wid < _TILES - 1)
    def _():
        pltpu.sync_copy(keys_hbm.at[pl.ds(base, _PERT)], buf)


def _bin1(v):
    return jnp.clip((v >> 13) - (_BASE13 - 1), 0, _NB1 - 1)


def _sc_hist1_body(keys_hbm, hist_hbm, buf, lh):
    wid = _wid()
    zeros16 = jnp.zeros((16,), jnp.int32)

    def z(i, _):
        lh[pl.ds(i * 16, 16)] = zeros16
        return 0

    lax.fori_loop(0, _NB1, z, 0)
    _load_chunk(keys_hbm, buf, wid)
    iota = lax.iota(jnp.int32, 16)
    ones = jnp.ones((16,), jnp.int32)

    def h(i, _):
        v = buf[pl.ds(i * 16, 16)]
        idx = iota * _NB1 + _bin1(v)
        plsc.addupdate_scatter(lh, [idx], ones)
        return 0

    lax.fori_loop(0, _VREGS, h, 0)

    def r(g, _):
        acc = lh[pl.ds(g * 16, 16)]
        for l in range(1, 16):
            acc = acc + lh[pl.ds(l * _NB1 + g * 16, 16)]
        buf[pl.ds(g * 16, 16)] = acc
        return 0

    lax.fori_loop(0, _NB1 // 16, r, 0)
    pltpu.sync_copy(buf.at[pl.ds(0, _NB1)], hist_hbm.at[wid])


def _sc_hist1(keys_flat):
    return pl.kernel(
        _sc_hist1_body,
        out_type=jax.ShapeDtypeStruct((_TILES, _NB1), jnp.int32),
        mesh=_sc_mesh(),
        compiler_params=pltpu.CompilerParams(needs_layout_passes=False),
        scratch_types=[
            pltpu.VMEM((_PERT,), jnp.int32),
            pltpu.VMEM((16 * _NB1,), jnp.int32),
        ],
    )(keys_flat)


def _sc_hist2_body(keys_hbm, b1_hbm, hist_hbm, buf, lh, bv):
    wid = _wid()
    zeros16 = jnp.zeros((16,), jnp.int32)

    def z(i, _):
        lh[pl.ds(i * 16, 16)] = zeros16
        return 0

    lax.fori_loop(0, 256, z, 0)
    pltpu.sync_copy(b1_hbm.at[0], bv)
    _load_chunk(keys_hbm, buf, wid)
    iota = lax.iota(jnp.int32, 16)
    ones = jnp.ones((16,), jnp.int32)
    vb1 = bv[pl.ds(0, 16)]

    def h(i, _):
        v = buf[pl.ds(i * 16, 16)]
        m = _bin1(v) == vb1
        idx = iota * 256 + ((v >> 5) & 0xFF)
        plsc.addupdate_scatter(lh, [idx], ones, mask=m)
        return 0

    lax.fori_loop(0, _VREGS, h, 0)

    def r(g, _):
        acc = lh[pl.ds(g * 16, 16)]
        for l in range(1, 16):
            acc = acc + lh[pl.ds(l * 256 + g * 16, 16)]
        buf[pl.ds(g * 16, 16)] = acc
        return 0

    lax.fori_loop(0, 16, r, 0)
    pltpu.sync_copy(buf.at[pl.ds(0, 256)], hist_hbm.at[wid])


def _sc_hist2(keys_flat, b1_vec):
    return pl.kernel(
        _sc_hist2_body,
        out_type=jax.ShapeDtypeStruct((_TILES, 256), jnp.int32),
        mesh=_sc_mesh(),
        compiler_params=pltpu.CompilerParams(needs_layout_passes=False),
        scratch_types=[
            pltpu.VMEM((_PERT,), jnp.int32),
            pltpu.VMEM((16 * 256,), jnp.int32),
            pltpu.VMEM((128,), jnp.int32),
        ],
    )(keys_flat, b1_vec)


def _sc_hist3_body(keys_hbm, b1_hbm, b2_hbm, hist_hbm, buf, lh, bv1, bv2):
    wid = _wid()
    zeros16 = jnp.zeros((16,), jnp.int32)

    def z(i, _):
        lh[pl.ds(i * 16, 16)] = zeros16
        return 0

    lax.fori_loop(0, 32, z, 0)
    pltpu.sync_copy(b1_hbm.at[0], bv1)
    pltpu.sync_copy(b2_hbm.at[0], bv2)
    _load_chunk(keys_hbm, buf, wid)
    iota = lax.iota(jnp.int32, 16)
    ones = jnp.ones((16,), jnp.int32)
    vb1 = bv1[pl.ds(0, 16)]
    vb2 = bv2[pl.ds(0, 16)]

    def h(i, _):
        v = buf[pl.ds(i * 16, 16)]
        m = (_bin1(v) == vb1) & (((v >> 5) & 0xFF) == vb2)
        idx = iota * 32 + (v & 0x1F)
        plsc.addupdate_scatter(lh, [idx], ones, mask=m)
        return 0

    lax.fori_loop(0, _VREGS, h, 0)

    def r(g, _):
        acc = lh[pl.ds(g * 16, 16)]
        for l in range(1, 16):
            acc = acc + lh[pl.ds(l * 32 + g * 16, 16)]
        buf[pl.ds(g * 16, 16)] = acc
        return 0

    lax.fori_loop(0, 2, r, 0)

    def zr(g, _):
        buf[pl.ds(32 + g * 16, 16)] = zeros16
        return 0

    lax.fori_loop(0, 6, zr, 0)
    pltpu.sync_copy(buf.at[pl.ds(0, 128)], hist_hbm.at[wid])


def _sc_hist3(keys_flat, b1_vec, b2_vec):
    return pl.kernel(
        _sc_hist3_body,
        out_type=jax.ShapeDtypeStruct((_TILES, 128), jnp.int32),
        mesh=_sc_mesh(),
        compiler_params=pltpu.CompilerParams(needs_layout_passes=False),
        scratch_types=[
            pltpu.VMEM((_PERT,), jnp.int32),
            pltpu.VMEM((16 * 32,), jnp.int32),
            pltpu.VMEM((128,), jnp.int32),
            pltpu.VMEM((128,), jnp.int32),
        ],
    )(keys_flat, b1_vec, b2_vec)


def _suffix_excl(g):
    R, L = g.shape
    src = lax.broadcasted_iota(jnp.int32, (L, L), 0)
    dst = lax.broadcasted_iota(jnp.int32, (L, L), 1)
    upper = (src > dst).astype(jnp.float32)
    w = lax.dot_general(g, upper, (((1,), (0,)), ((), ())),
                        preferred_element_type=jnp.float32)
    t = jnp.sum(g, axis=1, keepdims=True)
    rs = lax.broadcasted_iota(jnp.int32, (R, R), 0)
    rd = lax.broadcasted_iota(jnp.int32, (R, R), 1)
    later_rows = (rd > rs).astype(jnp.float32)
    carry = lax.dot_general(later_rows, t, (((1,), (0,)), ((), ())),
                            preferred_element_type=jnp.float32)
    return w + carry


def _crossing(g, suf, target):
    R, L = g.shape
    mask = (suf < target) & (suf + g >= target)
    bi = (lax.broadcasted_iota(jnp.int32, (R, L), 0) * L +
          lax.broadcasted_iota(jnp.int32, (R, L), 1))
    bstar = jnp.sum(jnp.where(mask, bi, 0))
    above = jnp.sum(jnp.where(mask, suf, 0.0)).astype(jnp.int32)
    return bstar, above


def _r1_body(h_ref, out_ref):
    h = h_ref[...].astype(jnp.float32)
    g = jnp.sum(h, axis=0)
    suf = _suffix_excl(g)
    bstar, a1 = _crossing(g, suf, float(_KPRE))
    lane = lax.broadcasted_iota(jnp.int32, (1, 128), 1)
    out_ref[...] = jnp.where(lane == 0, bstar, jnp.where(lane == 1, a1, 0))


def _r2_body(h_ref, r1_ref, out_ref):
    h = h_ref[...].astype(jnp.float32)
    a1 = r1_ref[0, 1]
    g = jnp.sum(h, axis=0)
    suf = _suffix_excl(g)
    b2, a2 = _crossing(g, suf, (_KPRE - a1).astype(jnp.float32))
    lane = lax.broadcasted_iota(jnp.int32, (1, 128), 1)
    out_ref[...] = jnp.where(lane == 0, b2, jnp.where(lane == 1, a2, 0))


def _r3_body(h1_ref, h2_ref, h3_ref, r1_ref, r2_ref,
             kstar_ref, o_ref, quota_ref, ts_ref):
    b1s = r1_ref[0, 0]
    a1 = r1_ref[0, 1]
    b2s = r2_ref[0, 0]
    a2 = r2_ref[0, 1]
    h3 = h3_ref[...].astype(jnp.float32)
    g3 = jnp.sum(h3, axis=0, keepdims=True)
    suf3 = _suffix_excl(g3)
    need2 = (_KPRE - a1 - a2).astype(jnp.float32)
    b3s, a3 = _crossing(g3, suf3, need2)
    a_total = a1 + a2 + a3
    tcnt = _KPRE - a_total

    key_normal = ((b1s - 1 + _BASE13) << 13) | (b2s << 5) | b3s
    key_bin0 = jnp.where(b2s == 0xFF, jnp.int32(-1065353217),
                         jnp.int32(_INT_MIN))
    kstar = jnp.where(b1s >= 1, key_normal, key_bin0)

    h1 = h1_ref[...].astype(jnp.float32)
    bi1 = (lax.broadcasted_iota(jnp.int32, (36, 128), 0) * 128 +
           lax.broadcasted_iota(jnp.int32, (36, 128), 1))
    g1_t = jnp.sum(h1 * (bi1 > b1s).astype(jnp.float32)[None], axis=(1, 2))
    h2 = h2_ref[...].astype(jnp.float32)
    bi2 = (lax.broadcasted_iota(jnp.int32, (2, 128), 0) * 128 +
           lax.broadcasted_iota(jnp.int32, (2, 128), 1))
    g2_t = jnp.sum(h2 * (bi2 > b2s).astype(jnp.float32)[None], axis=(1, 2))
    bi3 = lax.broadcasted_iota(jnp.int32, (_TILES, 128), 1)
    g3_t = jnp.sum(h3 * (bi3 > b3s).astype(jnp.float32), axis=1)
    c_t = jnp.sum(h3 * (bi3 == b3s).astype(jnp.float32), axis=1)
    g_t = (g1_t + g2_t + g3_t).reshape(1, _TILES)
    c_t = c_t.reshape(1, _TILES)

    ts_i = lax.broadcasted_iota(jnp.int32, (_TILES, _TILES), 0)
    td_i = lax.broadcasted_iota(jnp.int32, (_TILES, _TILES), 1)
    lower = (ts_i < td_i).astype(jnp.float32)
    p_t = lax.dot_general(c_t, lower, (((1,), (0,)), ((), ())),
                          preferred_element_type=jnp.float32)
    tf = tcnt.astype(jnp.float32)
    quota_t = jnp.clip(tf - p_t, 0.0, c_t)

    def c16(x):
        return jnp.floor((x + 15.0) * (1.0 / 16.0)) * 16.0

    g16 = c16(g_t)
    q16 = c16(quota_t)
    o_t = lax.dot_general(g16, lower, (((1,), (0,)), ((), ())),
                          preferred_element_type=jnp.float32)
    tie_base = jnp.sum(g16)
    ts_t = tie_base + lax.dot_general(q16, lower, (((1,), (0,)), ((), ())),
                                      preferred_element_type=jnp.float32)
    fill_start = tie_base + jnp.sum(q16)

    kstar_ref[...] = jnp.broadcast_to(kstar, (1, 128))
    lanes = lax.broadcasted_iota(jnp.int32, (1, 128), 1)
    zpad = jnp.zeros((1, 128 - _TILES), jnp.float32)
    o_full = jnp.concatenate([o_t, zpad], axis=1).astype(jnp.int32)
    o_ref[...] = jnp.where(lanes == _TILES, fill_start.astype(jnp.int32),
                           o_full)
    quota_ref[...] = jnp.concatenate([quota_t, zpad], axis=1).astype(jnp.int32)
    ts_ref[...] = jnp.concatenate([ts_t, zpad], axis=1).astype(jnp.int32)


def _reduce_kernels(h1, h2, h3):
    h1_3d = h1.reshape(_TILES, 36, 128)
    h2_3d = h2.reshape(_TILES, 2, 128)
    r1 = pl.pallas_call(
        _r1_body, out_shape=jax.ShapeDtypeStruct((1, 128), jnp.int32),
    )(h1_3d)
    r2 = pl.pallas_call(
        _r2_body, out_shape=jax.ShapeDtypeStruct((1, 128), jnp.int32),
    )(h2_3d, r1)
    kstar, o_t, quota_t, ts_t = pl.pallas_call(
        _r3_body,
        out_shape=[
            jax.ShapeDtypeStruct((1, 16), jnp.int32),
            jax.ShapeDtypeStruct((1, _TILES), jnp.int32),
            jax.ShapeDtypeStruct((1, _TILES), jnp.int32),
            jax.ShapeDtypeStruct((1, _TILES), jnp.int32),
        ],
    )(h1_3d, h2_3d, h3, r1, r2)
    return r1, r2, kstar, o_t, quota_t, ts_t


def _sc_compact_body(keys_hbm, kvec_hbm, o_hbm, q_hbm, ts_hbm,
                     outk_hbm, outi_hbm,
                     buf, gbk, gbi, tbi, tbk, kv16, sco, scq, scts):
    wid = _wid()
    base = wid * _PERT
    pltpu.sync_copy(kvec_hbm.at[0], kv16)
    pltpu.sync_copy(o_hbm.at[0], sco)
    pltpu.sync_copy(q_hbm.at[0], scq)
    pltpu.sync_copy(ts_hbm.at[0], scts)
    _load_chunk(keys_hbm, buf, wid)
    iota = lax.iota(jnp.int32, 16)
    kv = kv16[pl.ds(0, 16)]

    def scal(vec_ref, which):
        half = which // 16
        lane = which % 16
        vec = vec_ref[pl.ds(half * 16, 16)]
        return lax.reduce_max(jnp.where(iota == lane, vec, 0), axes=(0,))

    def step(i, carry):
        ng, nt = carry
        v = buf[pl.ds(i * 16, 16)]
        gm = v > kv
        tm = v == kv
        idxv = (base + i * 16) + iota
        plsc.store_compressed(gbk.at[pl.ds(ng, 16)], v, mask=gm)
        plsc.store_compressed(gbi.at[pl.ds(ng, 16)], idxv, mask=gm)
        tm2 = tm & (nt < 2032)
        plsc.store_compressed(tbi.at[pl.ds(nt, 16)], idxv, mask=tm2)
        cg = lax.reduce_max(plsc.all_reduce_population_count(gm), axes=(0,))
        ct = lax.reduce_max(plsc.all_reduce_population_count(tm2), axes=(0,))
        return ng + cg, nt + ct

    ng, nt = lax.fori_loop(0, _VREGS, step, (jnp.int32(0), jnp.int32(0)))

    my_o = scal(sco, wid)
    my_q = scal(scq, wid)
    my_ts = scal(scts, wid)
    mn16 = jnp.full((16,), _INT_MIN, jnp.int32)
    z16 = jnp.zeros((16,), jnp.int32)

    j0 = (ng // 16) * 16
    gk_tail = gbk[pl.ds(j0, 16)]
    gi_tail = gbi[pl.ds(j0, 16)]
    tailpos = j0 + iota
    gbk[pl.ds(j0, 16)] = jnp.where(tailpos < ng, gk_tail, mn16)
    gbi[pl.ds(j0, 16)] = jnp.where(tailpos < ng, gi_tail, z16)

    def cpg(j, _):
        off = pl.multiple_of(my_o + j * 16, 16)
        pltpu.sync_copy(gbk.at[pl.ds(j * 16, 16)],
                        outk_hbm.at[pl.ds(off, 16)])
        pltpu.sync_copy(gbi.at[pl.ds(j * 16, 16)],
                        outi_hbm.at[pl.ds(off, 16)])
        return 0

    lax.fori_loop(0, (ng + 15) // 16, cpg, 0)

    jq = (my_q // 16) * 16
    ti_tail = tbi[pl.ds(jq, 16)]
    qpos = jq + iota
    tbi[pl.ds(jq, 16)] = jnp.where(qpos < my_q, ti_tail, z16)

    def cpt(j, _):
        tpos = j * 16 + iota
        tbk[pl.ds(0, 16)] = jnp.where(tpos < my_q, kv, mn16)
        off = pl.multiple_of(my_ts + j * 16, 16)
        pltpu.sync_copy(tbk.at[pl.ds(0, 16)],
                        outk_hbm.at[pl.ds(off, 16)])
        pltpu.sync_copy(tbi.at[pl.ds(j * 16, 16)],
                        outi_hbm.at[pl.ds(off, 16)])
        return 0

    lax.fori_loop(0, (my_q + 15) // 16, cpt, 0)

    @pl.when(wid == _TILES - 1)
    def _():
        fill = scal(sco, _TILES)

        def fmn(j, _):
            tbk[pl.ds(j * 16, 16)] = mn16
            tbi[pl.ds(j * 16, 16)] = z16
            return 0

        lax.fori_loop(0, 32, fmn, 0)
        nbig = (4096 - fill) // 512

        def fbig(j, _):
            off = pl.multiple_of(fill + j * 512, 16)
            pltpu.sync_copy(tbk.at[pl.ds(0, 512)],
                            outk_hbm.at[pl.ds(off, 512)])
            pltpu.sync_copy(tbi.at[pl.ds(0, 512)],
                            outi_hbm.at[pl.ds(off, 512)])
            return 0

        lax.fori_loop(0, nbig, fbig, 0)
        start2 = fill + nbig * 512

        def fsm(j, _):
            off = pl.multiple_of(start2 + j * 16, 16)
            pltpu.sync_copy(tbk.at[pl.ds(0, 16)],
                            outk_hbm.at[pl.ds(off, 16)])
            pltpu.sync_copy(tbi.at[pl.ds(0, 16)],
                            outi_hbm.at[pl.ds(off, 16)])
            return 0

        lax.fori_loop(0, (4096 - start2) // 16, fsm, 0)


def _sc_compact(keys_flat, kstar_vec, o_t, quota_t, ts_t):
    return pl.kernel(
        _sc_compact_body,
        out_type=[
            jax.ShapeDtypeStruct((4096,), jnp.int32),
            jax.ShapeDtypeStruct((4096,), jnp.int32),
        ],
        mesh=_sc_mesh(),
        compiler_params=pltpu.CompilerParams(needs_layout_passes=False),
        scratch_types=[
            pltpu.VMEM((_PERT,), jnp.int32),
            pltpu.VMEM((2064,), jnp.int32),
            pltpu.VMEM((2064,), jnp.int32),
            pltpu.VMEM((2064,), jnp.int32),
            pltpu.VMEM((512,), jnp.int32),
            pltpu.VMEM((128,), jnp.int32),
            pltpu.VMEM((128,), jnp.int32),
            pltpu.VMEM((128,), jnp.int32),
            pltpu.VMEM((128,), jnp.int32),
        ],
    )(keys_flat, kstar_vec, o_t, quota_t, ts_t)


def _sort_body(k_ref, v_ref, ok_ref, ov_ref):
    keys = k_ref[...]
    vals = v_ref[...]
    i = lax.broadcasted_iota(jnp.int32, (1, 4096), 1)
    for size_p in range(1, 13):
        size = 1 << size_p
        dirm = (i & size) == 0
        for j_p in range(size_p - 1, -1, -1):
            j = 1 << j_p
            upperm = (i & j) != 0
            pk = jnp.where(upperm, pltpu.roll(keys, j, 1),
                           pltpu.roll(keys, 4096 - j, 1))
            pv = jnp.where(upperm, pltpu.roll(vals, j, 1),
                           pltpu.roll(vals, 4096 - j, 1))
            pf = (pk > keys) | ((pk == keys) & (pv < vals))
            take = pf ^ upperm ^ dirm ^ True
            keys = jnp.where(take, pk, keys)
            vals = jnp.where(take, pv, vals)
    ok_ref[...] = keys
    ov_ref[...] = vals


def _sort4096(keys, vals):
    return pl.pallas_call(
        _sort_body,
        out_shape=[jax.ShapeDtypeStruct((1, 4096), jnp.int32)] * 2,
    )(keys.reshape(1, 4096), vals.reshape(1, 4096))


_KP = 2048
_NB = 128
_NBLK = _KP // _NB


def _nms_body(x1c, y1c, x2c, y2c, x1r, y1r, x2r, y2r, keep_ref,
              iou_s, bb_s, kb_s):
    ax1 = x1r[...]
    ay1 = y1r[...]
    ax2 = x2r[...]
    ay2 = y2r[...]
    area_r = jnp.maximum(ax2 - ax1, 0.0) * jnp.maximum(ay2 - ay1, 0.0)
    keep_ref[...] = jnp.ones((1, _KP), jnp.float32)
    lane = lax.broadcasted_iota(jnp.int32, (1, _NB), 1)
    col = lax.broadcasted_iota(jnp.int32, (1, _KP), 1)

    for bi in range(_NBLK):
        s = bi * _NB
        w = _KP - s
        xb1 = x1c[s:s + _NB, :]
        yb1 = y1c[s:s + _NB, :]
        xb2 = x2c[s:s + _NB, :]
        yb2 = y2c[s:s + _NB, :]
        area_b = jnp.maximum(xb2 - xb1, 0.0) * jnp.maximum(yb2 - yb1, 0.0)
        xx1 = jnp.maximum(xb1, ax1[:, s:])
        yy1 = jnp.maximum(yb1, ay1[:, s:])
        xx2 = jnp.minimum(xb2, ax2[:, s:])
        yy2 = jnp.minimum(yb2, ay2[:, s:])
        inter = jnp.maximum(xx2 - xx1, 0.0) * jnp.maximum(yy2 - yy1, 0.0)
        iou = inter / (area_b + area_r[:, s:] - inter + 1e-9)
        hit = (iou > _NMS_THRESH).astype(jnp.float32)
        iou_s[:, 0:w] = hit
        bb_s[...] = hit[:, 0:_NB]
        kb_s[...] = keep_ref[0:1, s:s + _NB]

        def intra(i, _):
            row = bb_s[pl.ds(i, 1), :]
            kb = kb_s[...]
            ki = jnp.max(jnp.where(lane == i, kb, 0.0))
            sup = (row > 0.0) & (lane > i) & (ki > 0.0)
            kb_s[...] = jnp.where(sup, 0.0, kb)
            return 0

        lax.fori_loop(0, _NB, intra, 0, unroll=8)

        kept = kb_s[...]
        keep_ref[0:1, s:s + _NB] = kept
        if bi < _NBLK - 1:
            sup_all = lax.dot_general(
                kept, iou_s[:, _NB:w],
                dimension_numbers=(((1,), (0,)), ((), ())),
                preferred_element_type=jnp.float32)
            kl = keep_ref[0:1, s + _NB:]
            keep_ref[0:1, s + _NB:] = jnp.where(sup_all > 0.0, 0.0, kl)


def _nms_keep_pallas(boxes):
    pad = _KP - _KPRE
    b = jnp.pad(boxes, ((0, pad), (0, 0)))
    cols = [b[:, i:i + 1] for i in range(4)]
    rows = [b[:, i].reshape(1, _KP) for i in range(4)]
    keep = pl.pallas_call(
        _nms_body,
        out_shape=jax.ShapeDtypeStruct((1, _KP), jnp.float32),
        scratch_shapes=[
            pltpu.VMEM((_NB, _KP), jnp.float32),
            pltpu.VMEM((_NB, _NB), jnp.float32),
            pltpu.VMEM((1, _NB), jnp.float32),
        ],
    )(*cols, *rows)
    return keep[0, :_KPRE] > 0.0


@jax.jit
def kernel(class_logits, box_regression, proposals):
    keys, bx1, by1, bx2, by2 = _dense_stage(
        class_logits, box_regression, proposals)
    keys_flat = keys.reshape(-1)

    h1 = _sc_hist1(keys_flat)
    r1 = pl.pallas_call(
        _r1_body, out_shape=jax.ShapeDtypeStruct((1, 128), jnp.int32),
    )(h1.reshape(_TILES, 36, 128))
    b1_vec = jnp.broadcast_to(r1[0:1, 0:1], (1, 128))
    h2 = _sc_hist2(keys_flat, b1_vec)
    r2 = pl.pallas_call(
        _r2_body, out_shape=jax.ShapeDtypeStruct((1, 128), jnp.int32),
    )(h2.reshape(_TILES, 2, 128), r1)
    b2_vec = jnp.broadcast_to(r2[0:1, 0:1], (1, 128))
    h3 = _sc_hist3(keys_flat, b1_vec, b2_vec)
    kstar, o_t, quota_t, ts_t = pl.pallas_call(
        _r3_body,
        out_shape=[jax.ShapeDtypeStruct((1, 128), jnp.int32)] * 4,
    )(h1.reshape(_TILES, 36, 128), h2.reshape(_TILES, 2, 128), h3, r1, r2)

    outk, outi = _sc_compact(keys_flat, kstar, o_t, quota_t, ts_t)
    sk, sv = _sort4096(outk, outi)

    top_idx = sv[0, :_KPRE]
    skk = sk[0, :_KPRE]
    vbits = jnp.where(skk >= 0, skk, skk ^ 0x7FFFFFFF)
    top_vals = lax.bitcast_convert_type(vbits, jnp.float32)

    sel_x1 = bx1.reshape(-1)[top_idx]
    sel_y1 = by1.reshape(-1)[top_idx]
    sel_x2 = bx2.reshape(-1)[top_idx]
    sel_y2 = by2.reshape(-1)[top_idx]
    sel_boxes = jnp.stack([sel_x1, sel_y1, sel_x2, sel_y2], axis=1)
    sel_labels = (top_idx % 90) + 1
    max_coord = jnp.max(sel_boxes)
    offsets = sel_labels.astype(jnp.float32) * (max_coord + 1.0)
    keep = _nms_keep_pallas(sel_boxes + offsets[:, None])
    final_scores = jnp.where(keep & (top_vals > _SCORE_THRESH), top_vals, -1.0)
    fvals, fidx = jax.lax.top_k(final_scores, _DET)
    out = jnp.concatenate([sel_boxes[fidx], fvals[:, None]], axis=1)
    return jnp.where((fvals > _SCORE_THRESH)[:, None], out, 0.0)
